# baseline (speedup 1.0000x reference)
import functools

import jax
import jax.numpy as jnp
from jax import lax
from jax.experimental import pallas as pl
from jax.experimental.pallas import tpu as pltpu
from jax.experimental.pallas import tpu_sc as plsc

NN = 10000
NE = 8192
NH = 8192
NNZ = 32768
F = 128
MAP = 64

_NS = 16
_CHUNK = 128
_CNT_ROWS = 2 * NE + NN
_LOG2E = 1.4426950408889634


def _pad128(n):
    return -(-n // 128) * 128


def _emit_op(op, sid, tab_h, src_h, dst_h, zeros_h, out_h, sbuf, dbuf,
             rowss, gsems, ssems, acc):
    c = op["chunk"]
    nchunk = (op["k"] // _NS) // c
    const = bool(op.get("const_rows"))
    n_pad = op["n_pad"]
    rpt = n_pad // _NS
    zoff = 0
    while zoff < rpt:
        zn = min(_CHUNK, rpt - zoff)
        pltpu.sync_copy(zeros_h.at[pl.ds(0, zn)],
                        acc.at[pl.ds(sid * rpt + zoff, zn)])
        zoff += zn
    pltpu.sync_copy(dst_h.at[pl.ds(sid * nchunk, nchunk)], dbuf)
    if const:
        pltpu.sync_copy(src_h.at[pl.ds(sid * nchunk, 8)], sbuf)
        pltpu.async_copy(tab_h.at[sbuf.at[0]], rowss[0], gsems[0]).wait()
    else:
        pltpu.sync_copy(src_h.at[pl.ds(sid * nchunk, nchunk)], sbuf)
    plsc.subcore_barrier()
    gd = {}
    sd = {}
    if not const:
        gd[0] = pltpu.async_copy(tab_h.at[sbuf.at[0]], rowss[0], gsems[0])
    for t in range(nchunk):
        b = t % 2
        if t >= 1:
            sd[t - 1].wait()
        if not const and t + 1 < nchunk:
            gd[t + 1] = pltpu.async_copy(tab_h.at[sbuf.at[t + 1]],
                                         rowss[1 - b], gsems[1 - b])
        if not const:
            gd[t].wait()
        sd[t] = pltpu.async_copy(rowss[b] if not const else rowss[0],
                                 acc.at[dbuf.at[t]], ssems[b], add=True)
    sd[nchunk - 1].wait()
    plsc.subcore_barrier()
    pltpu.sync_copy(acc.at[pl.ds(sid * rpt, rpt)],
                    out_h.at[pl.ds(sid * rpt, rpt)])


def _sc_dual(op_a, op_b):
    ops = []
    for op in (op_a, op_b):
        o = dict(op)
        (o["k"],) = o["src"].shape
        o["w"] = o["table"].shape[1]
        o["n_pad"] = _pad128(o["n_out"])
        npt = o["k"] // _NS
        if "chunk" not in o:
            o["chunk"] = next(c for c in (64, 32)
                              if npt % c == 0 and (npt // c) % 8 == 0)
        o["src"] = o["src"].reshape(-1, o["chunk"])
        o["dst"] = o["dst"].reshape(-1, o["chunk"])
        o["zeros"] = jnp.zeros((_CHUNK, o["w"]), jnp.float32)
        ops.append(o)
    a, b = ops
    mesh = plsc.VectorSubcoreMesh(core_axis_name="c", subcore_axis_name="s")

    def body(ta, sa, da, za, tb, sb, db, zb, oa, ob,
             sbuf_a, dbuf_a, rows_a0, rows_a1,
             sbuf_b, dbuf_b, rows_b0, rows_b1,
             acc_a, acc_b, g0a, g1a, s0a, s1a, g0b, g1b, s0b, s1b):
        cid = lax.axis_index("c")
        sid = lax.axis_index("s")

        @pl.when(cid == 0)
        def _():
            _emit_op(a, sid, ta, sa, da, za, oa, sbuf_a, dbuf_a,
                     [rows_a0, rows_a1], [g0a, g1a], [s0a, s1a], acc_a)

        @pl.when(cid == 1)
        def _():
            _emit_op(b, sid, tb, sb, db, zb, ob, sbuf_b, dbuf_b,
                     [rows_b0, rows_b1], [g0b, g1b], [s0b, s1b], acc_b)

    f = pl.kernel(
        body,
        out_type=(jax.ShapeDtypeStruct((a["n_pad"], a["w"]), jnp.float32),
                  jax.ShapeDtypeStruct((b["n_pad"], b["w"]), jnp.float32)),
        mesh=mesh,
        compiler_params=pltpu.CompilerParams(use_tc_tiling_on_sc=False),
        scratch_types=[
            pltpu.VMEM((8 if a.get("const_rows") else a["k"] // _NS // a["chunk"],
                        a["chunk"]), jnp.int32),
            pltpu.VMEM((a["k"] // _NS // a["chunk"], a["chunk"]), jnp.int32),
            pltpu.VMEM((a["chunk"], a["w"]), jnp.float32),
            pltpu.VMEM((a["chunk"], a["w"]), jnp.float32),
            pltpu.VMEM((8 if b.get("const_rows") else b["k"] // _NS // b["chunk"],
                        b["chunk"]), jnp.int32),
            pltpu.VMEM((b["k"] // _NS // b["chunk"], b["chunk"]), jnp.int32),
            pltpu.VMEM((b["chunk"], b["w"]), jnp.float32),
            pltpu.VMEM((b["chunk"], b["w"]), jnp.float32),
            pltpu.VMEM_SHARED((a["n_pad"], a["w"]), jnp.float32),
            pltpu.VMEM_SHARED((b["n_pad"], b["w"]), jnp.float32),
        ] + [pltpu.SemaphoreType.DMA] * 8,
    )
    return f(a["table"], a["src"], a["dst"], a["zeros"],
             b["table"], b["src"], b["dst"], b["zeros"])


_RB = 512


def _mm_kernel(x_ref, w_ref, o_ref):
    o_ref[...] = jnp.dot(x_ref[...], w_ref[...], preferred_element_type=jnp.float32)


def _mm(x, w):
    n = x.shape[0]
    g = pl.cdiv(n, _RB)
    return pl.pallas_call(
        _mm_kernel,
        grid=(g,),
        in_specs=[pl.BlockSpec((_RB, x.shape[1]), lambda i: (i, 0)),
                  pl.BlockSpec(w.shape, lambda i: (0, 0))],
        out_specs=pl.BlockSpec((_RB, w.shape[1]), lambda i: (i, 0)),
        out_shape=jax.ShapeDtypeStruct((n, w.shape[1]), jnp.float32),
    )(x, w)


def _mm_dinv_kernel(x_ref, w_ref, cnt_ref, o_ref):
    dinv = lax.rsqrt(1.0 + cnt_ref[:, 0:1])
    o_ref[...] = jnp.dot(x_ref[...], w_ref[...],
                         preferred_element_type=jnp.float32) * dinv


def _mm_dinv(x, w, cnt, cblk0):
    n = x.shape[0]
    g = pl.cdiv(n, _RB)
    return pl.pallas_call(
        _mm_dinv_kernel,
        grid=(g,),
        in_specs=[pl.BlockSpec((_RB, x.shape[1]), lambda i: (i, 0)),
                  pl.BlockSpec(w.shape, lambda i: (0, 0)),
                  pl.BlockSpec((_RB, 16), lambda i: (cblk0 + i, 0))],
        out_specs=pl.BlockSpec((_RB, w.shape[1]), lambda i: (i, 0)),
        out_shape=jax.ShapeDtypeStruct((n, w.shape[1]), jnp.float32),
    )(x, w, cnt)


def _leaky(x):
    return jnp.where(x >= 0, x, 0.01 * x)


def _gcn2_kernel(sl_ref, sr_ref, z_ref, b_ref, w_ref, cnt_ref, o_ref):
    dinv = lax.rsqrt(1.0 + cnt_ref[:, 0:1])
    s = jnp.concatenate([sl_ref[...], sr_ref[...]], axis=1)
    h = _leaky(dinv * (s + z_ref[...]) + b_ref[...])
    o_ref[...] = jnp.dot(h, w_ref[...], preferred_element_type=jnp.float32) * dinv


def _gcn_layer2_z(sl, sr, z, b, w, cnt, cblk0):
    n = z.shape[0]
    g = pl.cdiv(n, _RB)
    return pl.pallas_call(
        _gcn2_kernel,
        grid=(g,),
        in_specs=[pl.BlockSpec((_RB, MAP), lambda i: (i, 0)),
                  pl.BlockSpec((_RB, MAP), lambda i: (i, 0)),
                  pl.BlockSpec((_RB, F), lambda i: (i, 0)),
                  pl.BlockSpec((1, F), lambda i: (0, 0)),
                  pl.BlockSpec((F, F), lambda i: (0, 0)),
                  pl.BlockSpec((_RB, 16), lambda i: (cblk0 + i, 0))],
        out_specs=pl.BlockSpec((_RB, F), lambda i: (i, 0)),
        out_shape=jax.ShapeDtypeStruct((n, F), jnp.float32),
    )(sl, sr, z, b.reshape(1, F), w, cnt)


def _binv_kernel(t_ref, cnt_ref, o_ref):
    c = cnt_ref[:, 0:1]
    o_ref[...] = jnp.where(c > 0, 1.0 / c, 0.0) * t_ref[...]


def _binv_scale(t, cnt, cblk0):
    g = NH // _RB
    return pl.pallas_call(
        _binv_kernel,
        grid=(g,),
        in_specs=[pl.BlockSpec((_RB, F), lambda i: (i, 0)),
                  pl.BlockSpec((_RB, 16), lambda i: (cblk0 + i, 0))],
        out_specs=pl.BlockSpec((_RB, F), lambda i: (i, 0)),
        out_shape=jax.ShapeDtypeStruct((NH, F), jnp.float32),
    )(t, cnt)


def _hyper_out_kernel(u_ref, b_ref, w_ref, cnt_ref, o_ref):
    c = cnt_ref[:, 0:1]
    dinv = jnp.where(c > 0, 1.0 / c, 0.0)
    g = _leaky(dinv * u_ref[...] + b_ref[...])
    o_ref[...] = jnp.dot(g, w_ref[...], preferred_element_type=jnp.float32)


def _hyper_layer_out(u, b, w, cnt, cblk0):
    g = NE // _RB
    return pl.pallas_call(
        _hyper_out_kernel,
        grid=(g,),
        in_specs=[pl.BlockSpec((_RB, F), lambda i: (i, 0)),
                  pl.BlockSpec((1, F), lambda i: (0, 0)),
                  pl.BlockSpec(w.shape, lambda i: (0, 0)),
                  pl.BlockSpec((_RB, 16), lambda i: (cblk0 + i, 0))],
        out_specs=pl.BlockSpec((_RB, w.shape[1]), lambda i: (i, 0)),
        out_shape=jax.ShapeDtypeStruct((NE, w.shape[1]), jnp.float32),
    )(u, b.reshape(1, F), w, cnt)


def _pq_kernel(sl_ref, sr_ref, z_ref, b_ref, wt_ref, wb_ref, cnt_ref, o_ref):
    dinv = lax.rsqrt(1.0 + cnt_ref[:, 0:1])
    s = jnp.concatenate([sl_ref[...], sr_ref[...]], axis=1)
    h2 = _leaky(dinv * (s + z_ref[...]) + b_ref[...])
    p = jnp.dot(h2, wt_ref[...], preferred_element_type=jnp.float32)
    q = jnp.dot(h2, wb_ref[...], preferred_element_type=jnp.float32)
    o_ref[...] = jnp.stack([p, q])


def _pq(sl, sr, z2, b2, wtop, wbot, cnt, cblk0):
    n = z2.shape[0]
    g = pl.cdiv(n, _RB)
    return pl.pallas_call(
        _pq_kernel,
        grid=(g,),
        in_specs=[pl.BlockSpec((_RB, MAP), lambda i: (i, 0)),
                  pl.BlockSpec((_RB, MAP), lambda i: (i, 0)),
                  pl.BlockSpec((_RB, F), lambda i: (i, 0)),
                  pl.BlockSpec((1, F), lambda i: (0, 0)),
                  pl.BlockSpec((F, MAP), lambda i: (0, 0)),
                  pl.BlockSpec((F, MAP), lambda i: (0, 0)),
                  pl.BlockSpec((_RB, 16), lambda i: (cblk0 + i, 0))],
        out_specs=pl.BlockSpec((2, _RB, MAP), lambda i: (0, i, 0)),
        out_shape=jax.ShapeDtypeStruct((2, n, MAP), jnp.float32),
    )(sl, sr, z2, b2.reshape(1, F), wtop, wbot, cnt)


def _maps_kernel(u_ref, hb_ref, ew_ref, eb_ref, nm_ref, nb_ref, cnt_ref,
                 emh_ref, nmh_ref):
    c = cnt_ref[:, 0:1]
    dinv = jnp.where(c > 0, 1.0 / c, 0.0)
    g2 = _leaky(dinv * u_ref[...] + hb_ref[...])
    em = jnp.dot(g2, ew_ref[...], preferred_element_type=jnp.float32) + eb_ref[...]
    emh = em * (lax.rsqrt(jnp.sum(em * em, axis=1, keepdims=True)) * _LOG2E)
    emh_ref[...] = emh.astype(jnp.bfloat16)
    nm = nm_ref[...] + nb_ref[...]
    nmh = nm * lax.rsqrt(jnp.sum(nm * nm, axis=1, keepdims=True))
    nmh_ref[...] = nmh.astype(jnp.bfloat16)


def _maps(u2, hgc_b2, edge_w, edge_b, nmp, node_b, cnt, cblk0):
    g = NE // _RB
    return pl.pallas_call(
        _maps_kernel,
        grid=(g,),
        in_specs=[pl.BlockSpec((_RB, F), lambda i: (i, 0)),
                  pl.BlockSpec((1, F), lambda i: (0, 0)),
                  pl.BlockSpec((F, MAP), lambda i: (0, 0)),
                  pl.BlockSpec((1, MAP), lambda i: (0, 0)),
                  pl.BlockSpec((_RB, MAP), lambda i: (i, 0)),
                  pl.BlockSpec((1, MAP), lambda i: (0, 0)),
                  pl.BlockSpec((_RB, 16), lambda i: (cblk0 + i, 0))],
        out_specs=[pl.BlockSpec((_RB, MAP), lambda i: (i, 0)),
                   pl.BlockSpec((_RB, MAP), lambda i: (i, 0))],
        out_shape=[jax.ShapeDtypeStruct((NE, MAP), jnp.bfloat16),
                   jax.ShapeDtypeStruct((NE, MAP), jnp.bfloat16)],
    )(u2, hgc_b2.reshape(1, F), edge_w, edge_b.reshape(1, MAP),
      nmp, node_b.reshape(1, MAP), cnt)


_CB = 512
_CG = NE // _CB


def _contrast_kernel(nmh_ref, emh_ref, o_ref, s0_ref, s1_ref, d_ref):
    i = pl.program_id(0)
    j = pl.program_id(1)
    a = nmh_ref[pl.ds(i * _CB, _CB), :]
    b = emh_ref[pl.ds(j * _CB, _CB), :]
    m = lax.dot_general(a, b, (((1,), (1,)), ((), ())),
                        preferred_element_type=jnp.float32)
    e = jnp.exp2(-jnp.abs(m))
    rs = jnp.sum(e, axis=1).reshape(_CB // 128, 128)
    cs = jnp.sum(e, axis=0).reshape(_CB // 128, 128)
    rsl = pl.ds(i * (_CB // 128), _CB // 128)
    csl = pl.ds(j * (_CB // 128), _CB // 128)

    @pl.when(j == 0)
    def _():
        s1_ref[rsl, :] = jnp.zeros((_CB // 128, 128), jnp.float32)

    @pl.when(i == 0)
    def _():
        s0_ref[csl, :] = jnp.zeros((_CB // 128, 128), jnp.float32)

    s1_ref[rsl, :] += rs
    s0_ref[csl, :] += cs

    @pl.when(i == j)
    def _():
        r = lax.broadcasted_iota(jnp.int32, (_CB, _CB), 0)
        c = lax.broadcasted_iota(jnp.int32, (_CB, _CB), 1)
        diag = jnp.sum(jnp.where(r == c, m, 0.0), axis=1)
        d_ref[rsl, :] = diag.reshape(_CB // 128, 128)

    @pl.when((i == _CG - 1) & (j == _CG - 1))
    def _():
        o_ref[...] = (jnp.abs(d_ref[...]) * (1.0 / _LOG2E) - jnp.log(2.0)
                      + jnp.log(s0_ref[...] + s1_ref[...]))


def _contrast(nmh, emh):
    out = pl.pallas_call(
        _contrast_kernel,
        grid=(_CG, _CG),
        in_specs=[pl.BlockSpec((NE, MAP), lambda i, j: (0, 0)),
                  pl.BlockSpec((NE, MAP), lambda i, j: (0, 0))],
        out_specs=pl.BlockSpec((NE // 128, 128), lambda i, j: (0, 0)),
        out_shape=jax.ShapeDtypeStruct((NE // 128, 128), jnp.float32),
        scratch_shapes=[pltpu.VMEM((NE // 128, 128), jnp.float32),
                        pltpu.VMEM((NE // 128, 128), jnp.float32),
                        pltpu.VMEM((NE // 128, 128), jnp.float32)],
    )(nmh, emh)
    return out.reshape(NE)


def kernel(nodes_feature, edges_feature, edge_index, hyperedge_index,
           gcn_w1, gcn_b1, gcn_w2, gcn_b2,
           hgc_w1, hgc_b1, hgc_w2, hgc_b2,
           node_w, node_b, edge_w, edge_b):
    ei0 = edge_index[0]
    ei1 = edge_index[1]
    hi0 = hyperedge_index[0]
    hi1 = hyperedge_index[1]

    x1e = _mm(edges_feature, hgc_w1)

    ones_tab = jnp.ones((8, 16), jnp.float32)
    dump = jnp.full((81920 - (2 * NNZ + NE),), _pad128(_CNT_ROWS) - 1,
                    jnp.int32)
    cdst = jnp.concatenate([hi0, NE + hi1, 2 * NE + ei1, dump])
    cnt, t1 = _sc_dual(
        dict(table=ones_tab, src=jnp.zeros_like(cdst), dst=cdst,
             n_out=_CNT_ROWS, const_rows=True, chunk=128),
        dict(table=x1e, src=hi0, dst=hi1, n_out=NH))
    dblk = 0
    bblk = NE // _RB
    gblk = 2 * NE // _RB

    z1 = _mm_dinv(nodes_feature, gcn_w1, cnt, gblk)
    he1 = _binv_scale(t1, cnt, bblk)

    z1h = z1.reshape(2 * NN, MAP)
    s1l, s1r = _sc_dual(
        dict(table=z1h, src=2 * ei0, dst=ei1, n_out=NN),
        dict(table=z1h, src=2 * ei0 + 1, dst=ei1, n_out=NN))

    z2 = _gcn_layer2_z(s1l, s1r, z1, gcn_b1, gcn_w2, cnt, gblk)

    z2h = z2.reshape(2 * NN, MAP)
    u1, s2l = _sc_dual(
        dict(table=he1, src=hi1, dst=hi0, n_out=NE, chunk=32),
        dict(table=z2h, src=2 * ei0, dst=ei1, n_out=NN))

    x2e = _hyper_layer_out(u1, hgc_b1, hgc_w2, cnt, dblk)

    t2, s2r = _sc_dual(
        dict(table=x2e, src=hi0, dst=hi1, n_out=NH, chunk=32),
        dict(table=z2h, src=2 * ei0 + 1, dst=ei1, n_out=NN))

    pq = _pq(s2l, s2r, z2, gcn_b2, node_w[:F], node_w[F:], cnt, gblk)
    he2 = _binv_scale(t2, cnt, bblk)

    iota_e = jnp.arange(NE, dtype=jnp.int32)
    nmsrc = jnp.concatenate([ei0, NN + ei1])
    nmdst = jnp.concatenate([iota_e, iota_e])
    u2, nmp = _sc_dual(
        dict(table=he2, src=hi1, dst=hi0, n_out=NE),
        dict(table=pq.reshape(2 * NN, MAP), src=nmsrc, dst=nmdst, n_out=NE))

    emh, nmh = _maps(u2, hgc_b2, edge_w, edge_b, nmp, node_b, cnt, dblk)
    return _contrast(nmh, emh)


# speedup vs baseline: 1.1543x; 1.1543x over previous
import functools

import jax
import jax.numpy as jnp
from jax import lax
from jax.experimental import pallas as pl
from jax.experimental.pallas import tpu as pltpu
from jax.experimental.pallas import tpu_sc as plsc

NN = 10000
NE = 8192
NH = 8192
NNZ = 32768
F = 128
MAP = 64

_NS = 16
_CHUNK = 128
_CNT_ROWS = 2 * NE + NN
_LOG2E = 1.4426950408889634


def _pad128(n):
    return -(-n // 128) * 128


def _emit_op(op, sid, tab_h, src_h, dst_h, zeros_h, out_h, sbuf, dbuf,
             rowss, gsems, ssems, acc):
    c = op["chunk"]
    nchunk = (op["k"] // _NS) // c
    const = bool(op.get("const_rows"))
    n_pad = op["n_pad"]
    rpt = n_pad // _NS
    zoff = 0
    while zoff < rpt:
        zn = min(_CHUNK, rpt - zoff)
        pltpu.sync_copy(zeros_h.at[pl.ds(0, zn)],
                        acc.at[pl.ds(sid * rpt + zoff, zn)])
        zoff += zn
    pltpu.sync_copy(dst_h.at[pl.ds(sid * nchunk, nchunk)], dbuf)
    if const:
        pltpu.sync_copy(src_h.at[pl.ds(sid * nchunk, 8)], sbuf)
        pltpu.async_copy(tab_h.at[sbuf.at[0]], rowss[0], gsems[0]).wait()
    else:
        pltpu.sync_copy(src_h.at[pl.ds(sid * nchunk, nchunk)], sbuf)
    plsc.subcore_barrier()
    gd = {}
    sd = {}
    if not const:
        gd[0] = pltpu.async_copy(tab_h.at[sbuf.at[0]], rowss[0], gsems[0])
    for t in range(nchunk):
        b = t % 2
        if t >= 1:
            sd[t - 1].wait()
        if not const and t + 1 < nchunk:
            gd[t + 1] = pltpu.async_copy(tab_h.at[sbuf.at[t + 1]],
                                         rowss[1 - b], gsems[1 - b])
        if not const:
            gd[t].wait()
        sd[t] = pltpu.async_copy(rowss[b] if not const else rowss[0],
                                 acc.at[dbuf.at[t]], ssems[b], add=True)
    sd[nchunk - 1].wait()
    plsc.subcore_barrier()
    pltpu.sync_copy(acc.at[pl.ds(sid * rpt, rpt)],
                    out_h.at[pl.ds(sid * rpt, rpt)])


def _sc_dual(op_a, op_b):
    ops = []
    for op in (op_a, op_b):
        o = dict(op)
        (o["k"],) = o["src"].shape
        o["w"] = o["table"].shape[1]
        o["n_pad"] = _pad128(o["n_out"])
        npt = o["k"] // _NS
        if "chunk" not in o:
            o["chunk"] = next(c for c in (64, 32)
                              if npt % c == 0 and (npt // c) % 8 == 0)
        o["src"] = o["src"].reshape(-1, o["chunk"])
        o["dst"] = o["dst"].reshape(-1, o["chunk"])
        o["zeros"] = jnp.zeros((_CHUNK, o["w"]), jnp.float32)
        ops.append(o)
    a, b = ops
    mesh = plsc.VectorSubcoreMesh(core_axis_name="c", subcore_axis_name="s")

    def body(ta, sa, da, za, tb, sb, db, zb, oa, ob,
             sbuf_a, dbuf_a, rows_a0, rows_a1,
             sbuf_b, dbuf_b, rows_b0, rows_b1,
             acc_a, acc_b, g0a, g1a, s0a, s1a, g0b, g1b, s0b, s1b):
        cid = lax.axis_index("c")
        sid = lax.axis_index("s")

        @pl.when(cid == 0)
        def _():
            _emit_op(a, sid, ta, sa, da, za, oa, sbuf_a, dbuf_a,
                     [rows_a0, rows_a1], [g0a, g1a], [s0a, s1a], acc_a)

        @pl.when(cid == 1)
        def _():
            _emit_op(b, sid, tb, sb, db, zb, ob, sbuf_b, dbuf_b,
                     [rows_b0, rows_b1], [g0b, g1b], [s0b, s1b], acc_b)

    f = pl.kernel(
        body,
        out_type=(jax.ShapeDtypeStruct((a["n_pad"], a["w"]), jnp.float32),
                  jax.ShapeDtypeStruct((b["n_pad"], b["w"]), jnp.float32)),
        mesh=mesh,
        compiler_params=pltpu.CompilerParams(use_tc_tiling_on_sc=False),
        scratch_types=[
            pltpu.VMEM((8 if a.get("const_rows") else a["k"] // _NS // a["chunk"],
                        a["chunk"]), jnp.int32),
            pltpu.VMEM((a["k"] // _NS // a["chunk"], a["chunk"]), jnp.int32),
            pltpu.VMEM((a["chunk"], a["w"]), jnp.float32),
            pltpu.VMEM((a["chunk"], a["w"]), jnp.float32),
            pltpu.VMEM((8 if b.get("const_rows") else b["k"] // _NS // b["chunk"],
                        b["chunk"]), jnp.int32),
            pltpu.VMEM((b["k"] // _NS // b["chunk"], b["chunk"]), jnp.int32),
            pltpu.VMEM((b["chunk"], b["w"]), jnp.float32),
            pltpu.VMEM((b["chunk"], b["w"]), jnp.float32),
            pltpu.VMEM_SHARED((a["n_pad"], a["w"]), jnp.float32),
            pltpu.VMEM_SHARED((b["n_pad"], b["w"]), jnp.float32),
        ] + [pltpu.SemaphoreType.DMA] * 8,
    )
    return f(a["table"], a["src"], a["dst"], a["zeros"],
             b["table"], b["src"], b["dst"], b["zeros"])


_RB = 512


def _mm_kernel(x_ref, w_ref, o_ref):
    o_ref[...] = jnp.dot(x_ref[...], w_ref[...], preferred_element_type=jnp.float32)


def _mm(x, w):
    n = x.shape[0]
    g = pl.cdiv(n, _RB)
    return pl.pallas_call(
        _mm_kernel,
        grid=(g,),
        in_specs=[pl.BlockSpec((_RB, x.shape[1]), lambda i: (i, 0)),
                  pl.BlockSpec(w.shape, lambda i: (0, 0))],
        out_specs=pl.BlockSpec((_RB, w.shape[1]), lambda i: (i, 0)),
        out_shape=jax.ShapeDtypeStruct((n, w.shape[1]), jnp.float32),
    )(x, w)


def _mm_dinv_kernel(x_ref, w_ref, cnt_ref, o_ref):
    dinv = lax.rsqrt(1.0 + cnt_ref[:, 0:1])
    o_ref[...] = jnp.dot(x_ref[...], w_ref[...],
                         preferred_element_type=jnp.float32) * dinv


def _mm_dinv(x, w, cnt, cblk0):
    n = x.shape[0]
    g = pl.cdiv(n, _RB)
    return pl.pallas_call(
        _mm_dinv_kernel,
        grid=(g,),
        in_specs=[pl.BlockSpec((_RB, x.shape[1]), lambda i: (i, 0)),
                  pl.BlockSpec(w.shape, lambda i: (0, 0)),
                  pl.BlockSpec((_RB, 16), lambda i: (cblk0 + i, 0))],
        out_specs=pl.BlockSpec((_RB, w.shape[1]), lambda i: (i, 0)),
        out_shape=jax.ShapeDtypeStruct((n, w.shape[1]), jnp.float32),
    )(x, w, cnt)


def _leaky(x):
    return jnp.where(x >= 0, x, 0.01 * x)


def _gcn2_kernel(sl_ref, sr_ref, z_ref, b_ref, w_ref, cnt_ref, o_ref):
    dinv = lax.rsqrt(1.0 + cnt_ref[:, 0:1])
    s = jnp.concatenate([sl_ref[...], sr_ref[...]], axis=1)
    h = _leaky(dinv * (s + z_ref[...]) + b_ref[...])
    o_ref[...] = jnp.dot(h, w_ref[...], preferred_element_type=jnp.float32) * dinv


def _gcn_layer2_z(sl, sr, z, b, w, cnt, cblk0):
    n = z.shape[0]
    g = pl.cdiv(n, _RB)
    return pl.pallas_call(
        _gcn2_kernel,
        grid=(g,),
        in_specs=[pl.BlockSpec((_RB, MAP), lambda i: (i, 0)),
                  pl.BlockSpec((_RB, MAP), lambda i: (i, 0)),
                  pl.BlockSpec((_RB, F), lambda i: (i, 0)),
                  pl.BlockSpec((1, F), lambda i: (0, 0)),
                  pl.BlockSpec((F, F), lambda i: (0, 0)),
                  pl.BlockSpec((_RB, 16), lambda i: (cblk0 + i, 0))],
        out_specs=pl.BlockSpec((_RB, F), lambda i: (i, 0)),
        out_shape=jax.ShapeDtypeStruct((n, F), jnp.float32),
    )(sl, sr, z, b.reshape(1, F), w, cnt)


def _binv_kernel(t_ref, cnt_ref, o_ref):
    c = cnt_ref[:, 0:1]
    o_ref[...] = jnp.where(c > 0, 1.0 / c, 0.0) * t_ref[...]


def _binv_scale(t, cnt, cblk0):
    g = NH // _RB
    return pl.pallas_call(
        _binv_kernel,
        grid=(g,),
        in_specs=[pl.BlockSpec((_RB, F), lambda i: (i, 0)),
                  pl.BlockSpec((_RB, 16), lambda i: (cblk0 + i, 0))],
        out_specs=pl.BlockSpec((_RB, F), lambda i: (i, 0)),
        out_shape=jax.ShapeDtypeStruct((NH, F), jnp.float32),
    )(t, cnt)


def _hyper_out_kernel(u_ref, b_ref, w_ref, cnt_ref, o_ref):
    c = cnt_ref[:, 0:1]
    dinv = jnp.where(c > 0, 1.0 / c, 0.0)
    g = _leaky(dinv * u_ref[...] + b_ref[...])
    o_ref[...] = jnp.dot(g, w_ref[...], preferred_element_type=jnp.float32)


def _hyper_layer_out(u, b, w, cnt, cblk0):
    g = NE // _RB
    return pl.pallas_call(
        _hyper_out_kernel,
        grid=(g,),
        in_specs=[pl.BlockSpec((_RB, F), lambda i: (i, 0)),
                  pl.BlockSpec((1, F), lambda i: (0, 0)),
                  pl.BlockSpec(w.shape, lambda i: (0, 0)),
                  pl.BlockSpec((_RB, 16), lambda i: (cblk0 + i, 0))],
        out_specs=pl.BlockSpec((_RB, w.shape[1]), lambda i: (i, 0)),
        out_shape=jax.ShapeDtypeStruct((NE, w.shape[1]), jnp.float32),
    )(u, b.reshape(1, F), w, cnt)


def _pq_kernel(sl_ref, sr_ref, z_ref, b_ref, wt_ref, wb_ref, cnt_ref, o_ref):
    dinv = lax.rsqrt(1.0 + cnt_ref[:, 0:1])
    s = jnp.concatenate([sl_ref[...], sr_ref[...]], axis=1)
    h2 = _leaky(dinv * (s + z_ref[...]) + b_ref[...])
    p = jnp.dot(h2, wt_ref[...], preferred_element_type=jnp.float32)
    q = jnp.dot(h2, wb_ref[...], preferred_element_type=jnp.float32)
    o_ref[...] = jnp.stack([p, q])


def _pq(sl, sr, z2, b2, wtop, wbot, cnt, cblk0):
    n = z2.shape[0]
    g = pl.cdiv(n, _RB)
    return pl.pallas_call(
        _pq_kernel,
        grid=(g,),
        in_specs=[pl.BlockSpec((_RB, MAP), lambda i: (i, 0)),
                  pl.BlockSpec((_RB, MAP), lambda i: (i, 0)),
                  pl.BlockSpec((_RB, F), lambda i: (i, 0)),
                  pl.BlockSpec((1, F), lambda i: (0, 0)),
                  pl.BlockSpec((F, MAP), lambda i: (0, 0)),
                  pl.BlockSpec((F, MAP), lambda i: (0, 0)),
                  pl.BlockSpec((_RB, 16), lambda i: (cblk0 + i, 0))],
        out_specs=pl.BlockSpec((2, _RB, MAP), lambda i: (0, i, 0)),
        out_shape=jax.ShapeDtypeStruct((2, n, MAP), jnp.float32),
    )(sl, sr, z2, b2.reshape(1, F), wtop, wbot, cnt)


def _maps_kernel(u_ref, hb_ref, ew_ref, eb_ref, nm_ref, nb_ref, cnt_ref,
                 emh_ref, nmh_ref):
    c = cnt_ref[:, 0:1]
    dinv = jnp.where(c > 0, 1.0 / c, 0.0)
    g2 = _leaky(dinv * u_ref[...] + hb_ref[...])
    em = jnp.dot(g2, ew_ref[...], preferred_element_type=jnp.float32) + eb_ref[...]
    emh = em * (lax.rsqrt(jnp.sum(em * em, axis=1, keepdims=True)) * _LOG2E)
    emh_ref[...] = emh.astype(jnp.bfloat16)
    nm = nm_ref[...] + nb_ref[...]
    nmh = nm * lax.rsqrt(jnp.sum(nm * nm, axis=1, keepdims=True))
    nmh_ref[...] = nmh.astype(jnp.bfloat16)


def _maps(u2, hgc_b2, edge_w, edge_b, nmp, node_b, cnt, cblk0):
    g = NE // _RB
    return pl.pallas_call(
        _maps_kernel,
        grid=(g,),
        in_specs=[pl.BlockSpec((_RB, F), lambda i: (i, 0)),
                  pl.BlockSpec((1, F), lambda i: (0, 0)),
                  pl.BlockSpec((F, MAP), lambda i: (0, 0)),
                  pl.BlockSpec((1, MAP), lambda i: (0, 0)),
                  pl.BlockSpec((_RB, MAP), lambda i: (i, 0)),
                  pl.BlockSpec((1, MAP), lambda i: (0, 0)),
                  pl.BlockSpec((_RB, 16), lambda i: (cblk0 + i, 0))],
        out_specs=[pl.BlockSpec((_RB, MAP), lambda i: (i, 0)),
                   pl.BlockSpec((_RB, MAP), lambda i: (i, 0))],
        out_shape=[jax.ShapeDtypeStruct((NE, MAP), jnp.bfloat16),
                   jax.ShapeDtypeStruct((NE, MAP), jnp.bfloat16)],
    )(u2, hgc_b2.reshape(1, F), edge_w, edge_b.reshape(1, MAP),
      nmp, node_b.reshape(1, MAP), cnt)


_CB = 1024
_CG = NE // _CB


def _contrast_kernel(nmh_ref, emh_ref, o_ref, s0_ref, s1_ref, d_ref):
    i = pl.program_id(0)
    j = pl.program_id(1)
    a = nmh_ref[pl.ds(i * _CB, _CB), :]
    b = emh_ref[pl.ds(j * _CB, _CB), :]
    m = lax.dot_general(a, b, (((1,), (1,)), ((), ())),
                        preferred_element_type=jnp.float32)
    e = jnp.exp2(-jnp.abs(m))
    ones = jnp.ones((_CB,), jnp.float32)
    rs = lax.dot_general(e, ones, (((1,), (0,)), ((), ())),
                         preferred_element_type=jnp.float32
                         ).reshape(_CB // 128, 128)
    cs = lax.dot_general(ones, e, (((0,), (0,)), ((), ())),
                         preferred_element_type=jnp.float32
                         ).reshape(_CB // 128, 128)
    rsl = pl.ds(i * (_CB // 128), _CB // 128)
    csl = pl.ds(j * (_CB // 128), _CB // 128)

    @pl.when(j == 0)
    def _():
        s1_ref[rsl, :] = jnp.zeros((_CB // 128, 128), jnp.float32)

    @pl.when(i == 0)
    def _():
        s0_ref[csl, :] = jnp.zeros((_CB // 128, 128), jnp.float32)

    s1_ref[rsl, :] += rs
    s0_ref[csl, :] += cs

    @pl.when(i == j)
    def _():
        r = lax.broadcasted_iota(jnp.int32, (_CB, _CB), 0)
        c = lax.broadcasted_iota(jnp.int32, (_CB, _CB), 1)
        diag = jnp.sum(jnp.where(r == c, m, 0.0), axis=1)
        d_ref[rsl, :] = diag.reshape(_CB // 128, 128)

    @pl.when((i == _CG - 1) & (j == _CG - 1))
    def _():
        o_ref[...] = (jnp.abs(d_ref[...]) * (1.0 / _LOG2E) - jnp.log(2.0)
                      + jnp.log(s0_ref[...] + s1_ref[...]))


def _contrast(nmh, emh):
    out = pl.pallas_call(
        _contrast_kernel,
        grid=(_CG, _CG),
        in_specs=[pl.BlockSpec((NE, MAP), lambda i, j: (0, 0)),
                  pl.BlockSpec((NE, MAP), lambda i, j: (0, 0))],
        out_specs=pl.BlockSpec((NE // 128, 128), lambda i, j: (0, 0)),
        out_shape=jax.ShapeDtypeStruct((NE // 128, 128), jnp.float32),
        scratch_shapes=[pltpu.VMEM((NE // 128, 128), jnp.float32),
                        pltpu.VMEM((NE // 128, 128), jnp.float32),
                        pltpu.VMEM((NE // 128, 128), jnp.float32)],
    )(nmh, emh)
    return out.reshape(NE)


def kernel(nodes_feature, edges_feature, edge_index, hyperedge_index,
           gcn_w1, gcn_b1, gcn_w2, gcn_b2,
           hgc_w1, hgc_b1, hgc_w2, hgc_b2,
           node_w, node_b, edge_w, edge_b):
    ei0 = edge_index[0]
    ei1 = edge_index[1]
    hi0 = hyperedge_index[0]
    hi1 = hyperedge_index[1]

    x1e = _mm(edges_feature, hgc_w1)

    ones_tab = jnp.ones((8, 16), jnp.float32)
    cdst = jnp.concatenate([hi0, NE + hi1, 2 * NE + ei1])
    cnt, t1 = _sc_dual(
        dict(table=ones_tab, src=jnp.zeros_like(cdst), dst=cdst,
             n_out=_CNT_ROWS, const_rows=True),
        dict(table=x1e, src=hi0, dst=hi1, n_out=NH))
    dblk = 0
    bblk = NE // _RB
    gblk = 2 * NE // _RB

    z1 = _mm_dinv(nodes_feature, gcn_w1, cnt, gblk)
    he1 = _binv_scale(t1, cnt, bblk)

    z1h = z1.reshape(2 * NN, MAP)
    s1l, s1r = _sc_dual(
        dict(table=z1h, src=2 * ei0, dst=ei1, n_out=NN),
        dict(table=z1h, src=2 * ei0 + 1, dst=ei1, n_out=NN))

    z2 = _gcn_layer2_z(s1l, s1r, z1, gcn_b1, gcn_w2, cnt, gblk)

    z2h = z2.reshape(2 * NN, MAP)
    u1, s2l = _sc_dual(
        dict(table=he1, src=hi1, dst=hi0, n_out=NE, chunk=32),
        dict(table=z2h, src=2 * ei0, dst=ei1, n_out=NN))

    x2e = _hyper_layer_out(u1, hgc_b1, hgc_w2, cnt, dblk)

    t2, s2r = _sc_dual(
        dict(table=x2e, src=hi0, dst=hi1, n_out=NH, chunk=32),
        dict(table=z2h, src=2 * ei0 + 1, dst=ei1, n_out=NN))

    pq = _pq(s2l, s2r, z2, gcn_b2, node_w[:F], node_w[F:], cnt, gblk)
    he2 = _binv_scale(t2, cnt, bblk)

    iota_e = jnp.arange(NE, dtype=jnp.int32)
    nmsrc = jnp.concatenate([ei0, NN + ei1])
    nmdst = jnp.concatenate([iota_e, iota_e])
    u2, nmp = _sc_dual(
        dict(table=he2, src=hi1, dst=hi0, n_out=NE),
        dict(table=pq.reshape(2 * NN, MAP), src=nmsrc, dst=nmdst, n_out=NE))

    emh, nmh = _maps(u2, hgc_b2, edge_w, edge_b, nmp, node_b, cnt, dblk)
    return _contrast(nmh, emh)


# speedup vs baseline: 1.1598x; 1.0048x over previous
import functools

import jax
import jax.numpy as jnp
from jax import lax
from jax.experimental import pallas as pl
from jax.experimental.pallas import tpu as pltpu
from jax.experimental.pallas import tpu_sc as plsc

NN = 10000
NE = 8192
NH = 8192
NNZ = 32768
F = 128
MAP = 64

_NS = 16
_CHUNK = 128
_CNT_ROWS = 2 * NE + NN
_LOG2E = 1.4426950408889634


def _pad128(n):
    return -(-n // 128) * 128


def _emit_op(op, sid, tab_h, src_h, dst_h, zeros_h, out_h, sbuf, dbuf,
             rowss, gsems, ssems, acc):
    c = op["chunk"]
    nchunk = (op["k"] // _NS) // c
    const = bool(op.get("const_rows"))
    n_pad = op["n_pad"]
    rpt = n_pad // _NS
    zoff = 0
    while zoff < rpt:
        zn = min(_CHUNK, rpt - zoff)
        pltpu.sync_copy(zeros_h.at[pl.ds(0, zn)],
                        acc.at[pl.ds(sid * rpt + zoff, zn)])
        zoff += zn
    pltpu.sync_copy(dst_h.at[pl.ds(sid * nchunk, nchunk)], dbuf)
    if const:
        pltpu.sync_copy(src_h.at[pl.ds(sid * nchunk, 8)], sbuf)
        pltpu.async_copy(tab_h.at[sbuf.at[0]], rowss[0], gsems[0]).wait()
    else:
        pltpu.sync_copy(src_h.at[pl.ds(sid * nchunk, nchunk)], sbuf)
    plsc.subcore_barrier()
    gd = {}
    sd = {}
    if not const:
        gd[0] = pltpu.async_copy(tab_h.at[sbuf.at[0]], rowss[0], gsems[0])
    for t in range(nchunk):
        b = t % 2
        if t >= 1:
            sd[t - 1].wait()
        if not const and t + 1 < nchunk:
            gd[t + 1] = pltpu.async_copy(tab_h.at[sbuf.at[t + 1]],
                                         rowss[1 - b], gsems[1 - b])
        if not const:
            gd[t].wait()
        sd[t] = pltpu.async_copy(rowss[b] if not const else rowss[0],
                                 acc.at[dbuf.at[t]], ssems[b], add=True)
    sd[nchunk - 1].wait()
    plsc.subcore_barrier()
    pltpu.sync_copy(acc.at[pl.ds(sid * rpt, rpt)],
                    out_h.at[pl.ds(sid * rpt, rpt)])


def _sc_dual(op_a, op_b):
    ops = []
    for op in (op_a, op_b):
        o = dict(op)
        (o["k"],) = o["src"].shape
        o["w"] = o["table"].shape[1]
        o["n_pad"] = _pad128(o["n_out"])
        npt = o["k"] // _NS
        if "chunk" not in o:
            o["chunk"] = next(c for c in (64, 32)
                              if npt % c == 0 and (npt // c) % 8 == 0)
        o["src"] = o["src"].reshape(-1, o["chunk"])
        o["dst"] = o["dst"].reshape(-1, o["chunk"])
        o["zeros"] = jnp.zeros((_CHUNK, o["w"]), jnp.float32)
        ops.append(o)
    a, b = ops
    mesh = plsc.VectorSubcoreMesh(core_axis_name="c", subcore_axis_name="s")

    def body(ta, sa, da, za, tb, sb, db, zb, oa, ob,
             sbuf_a, dbuf_a, rows_a0, rows_a1,
             sbuf_b, dbuf_b, rows_b0, rows_b1,
             acc_a, acc_b, g0a, g1a, s0a, s1a, g0b, g1b, s0b, s1b):
        cid = lax.axis_index("c")
        sid = lax.axis_index("s")

        @pl.when(cid == 0)
        def _():
            _emit_op(a, sid, ta, sa, da, za, oa, sbuf_a, dbuf_a,
                     [rows_a0, rows_a1], [g0a, g1a], [s0a, s1a], acc_a)

        @pl.when(cid == 1)
        def _():
            _emit_op(b, sid, tb, sb, db, zb, ob, sbuf_b, dbuf_b,
                     [rows_b0, rows_b1], [g0b, g1b], [s0b, s1b], acc_b)

    f = pl.kernel(
        body,
        out_type=(jax.ShapeDtypeStruct((a["n_pad"], a["w"]), jnp.float32),
                  jax.ShapeDtypeStruct((b["n_pad"], b["w"]), jnp.float32)),
        mesh=mesh,
        compiler_params=pltpu.CompilerParams(use_tc_tiling_on_sc=False),
        scratch_types=[
            pltpu.VMEM((8 if a.get("const_rows") else a["k"] // _NS // a["chunk"],
                        a["chunk"]), jnp.int32),
            pltpu.VMEM((a["k"] // _NS // a["chunk"], a["chunk"]), jnp.int32),
            pltpu.VMEM((a["chunk"], a["w"]), jnp.float32),
            pltpu.VMEM((a["chunk"], a["w"]), jnp.float32),
            pltpu.VMEM((8 if b.get("const_rows") else b["k"] // _NS // b["chunk"],
                        b["chunk"]), jnp.int32),
            pltpu.VMEM((b["k"] // _NS // b["chunk"], b["chunk"]), jnp.int32),
            pltpu.VMEM((b["chunk"], b["w"]), jnp.float32),
            pltpu.VMEM((b["chunk"], b["w"]), jnp.float32),
            pltpu.VMEM_SHARED((a["n_pad"], a["w"]), jnp.float32),
            pltpu.VMEM_SHARED((b["n_pad"], b["w"]), jnp.float32),
        ] + [pltpu.SemaphoreType.DMA] * 8,
    )
    return f(a["table"], a["src"], a["dst"], a["zeros"],
             b["table"], b["src"], b["dst"], b["zeros"])


_RB = 512


def _mm_kernel(x_ref, w_ref, o_ref):
    o_ref[...] = jnp.dot(x_ref[...], w_ref[...], preferred_element_type=jnp.float32)


def _mm(x, w):
    n = x.shape[0]
    g = pl.cdiv(n, _RB)
    return pl.pallas_call(
        _mm_kernel,
        grid=(g,),
        in_specs=[pl.BlockSpec((_RB, x.shape[1]), lambda i: (i, 0)),
                  pl.BlockSpec(w.shape, lambda i: (0, 0))],
        out_specs=pl.BlockSpec((_RB, w.shape[1]), lambda i: (i, 0)),
        out_shape=jax.ShapeDtypeStruct((n, w.shape[1]), jnp.float32),
    )(x, w)


def _mm_dinv_kernel(x_ref, w_ref, cnt_ref, o_ref):
    dinv = lax.rsqrt(1.0 + cnt_ref[:, 0:1])
    o_ref[...] = jnp.dot(x_ref[...], w_ref[...],
                         preferred_element_type=jnp.float32) * dinv


def _mm_dinv(x, w, cnt, cblk0):
    n = x.shape[0]
    g = pl.cdiv(n, _RB)
    return pl.pallas_call(
        _mm_dinv_kernel,
        grid=(g,),
        in_specs=[pl.BlockSpec((_RB, x.shape[1]), lambda i: (i, 0)),
                  pl.BlockSpec(w.shape, lambda i: (0, 0)),
                  pl.BlockSpec((_RB, 16), lambda i: (cblk0 + i, 0))],
        out_specs=pl.BlockSpec((_RB, w.shape[1]), lambda i: (i, 0)),
        out_shape=jax.ShapeDtypeStruct((n, w.shape[1]), jnp.float32),
    )(x, w, cnt)


def _edge_clamp(i):
    return jnp.minimum(i, NE // _RB - 1)


def _leaky(x):
    return jnp.where(x >= 0, x, 0.01 * x)


def _gcn2_kernel(sl_ref, sr_ref, z_ref, b_ref, w_ref, cnt_ref,
                 t_ref, cntb_ref, o_ref, he_ref):
    dinv = lax.rsqrt(1.0 + cnt_ref[:, 0:1])
    s = jnp.concatenate([sl_ref[...], sr_ref[...]], axis=1)
    h = _leaky(dinv * (s + z_ref[...]) + b_ref[...])
    o_ref[...] = jnp.dot(h, w_ref[...], preferred_element_type=jnp.float32) * dinv
    c = cntb_ref[:, 0:1]
    he_ref[...] = jnp.where(c > 0, 1.0 / c, 0.0) * t_ref[...]


def _gcn_layer2_z(sl, sr, z, b, w, cnt, cblk0, t, tblk0):
    n = z.shape[0]
    g = pl.cdiv(n, _RB)
    return pl.pallas_call(
        _gcn2_kernel,
        grid=(g,),
        in_specs=[pl.BlockSpec((_RB, MAP), lambda i: (i, 0)),
                  pl.BlockSpec((_RB, MAP), lambda i: (i, 0)),
                  pl.BlockSpec((_RB, F), lambda i: (i, 0)),
                  pl.BlockSpec((1, F), lambda i: (0, 0)),
                  pl.BlockSpec((F, F), lambda i: (0, 0)),
                  pl.BlockSpec((_RB, 16), lambda i: (cblk0 + i, 0)),
                  pl.BlockSpec((_RB, F), lambda i: (_edge_clamp(i), 0)),
                  pl.BlockSpec((_RB, 16),
                               lambda i: (tblk0 + _edge_clamp(i), 0))],
        out_specs=[pl.BlockSpec((_RB, F), lambda i: (i, 0)),
                   pl.BlockSpec((_RB, F), lambda i: (_edge_clamp(i), 0))],
        out_shape=[jax.ShapeDtypeStruct((n, F), jnp.float32),
                   jax.ShapeDtypeStruct((NE, F), jnp.float32)],
    )(sl, sr, z, b.reshape(1, F), w, cnt, t, cnt)


def _hyper_out_kernel(u_ref, b_ref, w_ref, cnt_ref, o_ref):
    c = cnt_ref[:, 0:1]
    dinv = jnp.where(c > 0, 1.0 / c, 0.0)
    g = _leaky(dinv * u_ref[...] + b_ref[...])
    o_ref[...] = jnp.dot(g, w_ref[...], preferred_element_type=jnp.float32)


def _hyper_layer_out(u, b, w, cnt, cblk0):
    g = NE // _RB
    return pl.pallas_call(
        _hyper_out_kernel,
        grid=(g,),
        in_specs=[pl.BlockSpec((_RB, F), lambda i: (i, 0)),
                  pl.BlockSpec((1, F), lambda i: (0, 0)),
                  pl.BlockSpec(w.shape, lambda i: (0, 0)),
                  pl.BlockSpec((_RB, 16), lambda i: (cblk0 + i, 0))],
        out_specs=pl.BlockSpec((_RB, w.shape[1]), lambda i: (i, 0)),
        out_shape=jax.ShapeDtypeStruct((NE, w.shape[1]), jnp.float32),
    )(u, b.reshape(1, F), w, cnt)


def _pq_kernel(sl_ref, sr_ref, z_ref, b_ref, wt_ref, wb_ref, cnt_ref,
               t_ref, cntb_ref, o_ref, he_ref):
    dinv = lax.rsqrt(1.0 + cnt_ref[:, 0:1])
    s = jnp.concatenate([sl_ref[...], sr_ref[...]], axis=1)
    h2 = _leaky(dinv * (s + z_ref[...]) + b_ref[...])
    p = jnp.dot(h2, wt_ref[...], preferred_element_type=jnp.float32)
    q = jnp.dot(h2, wb_ref[...], preferred_element_type=jnp.float32)
    o_ref[...] = jnp.stack([p, q])
    c = cntb_ref[:, 0:1]
    he_ref[...] = jnp.where(c > 0, 1.0 / c, 0.0) * t_ref[...]


def _pq(sl, sr, z2, b2, wtop, wbot, cnt, cblk0, t, tblk0):
    n = z2.shape[0]
    g = pl.cdiv(n, _RB)
    return pl.pallas_call(
        _pq_kernel,
        grid=(g,),
        in_specs=[pl.BlockSpec((_RB, MAP), lambda i: (i, 0)),
                  pl.BlockSpec((_RB, MAP), lambda i: (i, 0)),
                  pl.BlockSpec((_RB, F), lambda i: (i, 0)),
                  pl.BlockSpec((1, F), lambda i: (0, 0)),
                  pl.BlockSpec((F, MAP), lambda i: (0, 0)),
                  pl.BlockSpec((F, MAP), lambda i: (0, 0)),
                  pl.BlockSpec((_RB, 16), lambda i: (cblk0 + i, 0)),
                  pl.BlockSpec((_RB, F), lambda i: (_edge_clamp(i), 0)),
                  pl.BlockSpec((_RB, 16),
                               lambda i: (tblk0 + _edge_clamp(i), 0))],
        out_specs=[pl.BlockSpec((2, _RB, MAP), lambda i: (0, i, 0)),
                   pl.BlockSpec((_RB, F), lambda i: (_edge_clamp(i), 0))],
        out_shape=[jax.ShapeDtypeStruct((2, n, MAP), jnp.float32),
                   jax.ShapeDtypeStruct((NE, F), jnp.float32)],
    )(sl, sr, z2, b2.reshape(1, F), wtop, wbot, cnt, t, cnt)


def _maps_kernel(u_ref, hb_ref, ew_ref, eb_ref, nm_ref, nb_ref, cnt_ref,
                 emh_ref, nmh_ref):
    c = cnt_ref[:, 0:1]
    dinv = jnp.where(c > 0, 1.0 / c, 0.0)
    g2 = _leaky(dinv * u_ref[...] + hb_ref[...])
    em = jnp.dot(g2, ew_ref[...], preferred_element_type=jnp.float32) + eb_ref[...]
    emh = em * (lax.rsqrt(jnp.sum(em * em, axis=1, keepdims=True)) * _LOG2E)
    emh_ref[...] = emh.astype(jnp.bfloat16)
    nm = nm_ref[...] + nb_ref[...]
    nmh = nm * lax.rsqrt(jnp.sum(nm * nm, axis=1, keepdims=True))
    nmh_ref[...] = nmh.astype(jnp.bfloat16)


def _maps(u2, hgc_b2, edge_w, edge_b, nmp, node_b, cnt, cblk0):
    g = NE // _RB
    return pl.pallas_call(
        _maps_kernel,
        grid=(g,),
        in_specs=[pl.BlockSpec((_RB, F), lambda i: (i, 0)),
                  pl.BlockSpec((1, F), lambda i: (0, 0)),
                  pl.BlockSpec((F, MAP), lambda i: (0, 0)),
                  pl.BlockSpec((1, MAP), lambda i: (0, 0)),
                  pl.BlockSpec((_RB, MAP), lambda i: (i, 0)),
                  pl.BlockSpec((1, MAP), lambda i: (0, 0)),
                  pl.BlockSpec((_RB, 16), lambda i: (cblk0 + i, 0))],
        out_specs=[pl.BlockSpec((_RB, MAP), lambda i: (i, 0)),
                   pl.BlockSpec((_RB, MAP), lambda i: (i, 0))],
        out_shape=[jax.ShapeDtypeStruct((NE, MAP), jnp.bfloat16),
                   jax.ShapeDtypeStruct((NE, MAP), jnp.bfloat16)],
    )(u2, hgc_b2.reshape(1, F), edge_w, edge_b.reshape(1, MAP),
      nmp, node_b.reshape(1, MAP), cnt)


_CB = 1024
_CG = NE // _CB


def _contrast_kernel(nmh_ref, emh_ref, o_ref, s0_ref, s1_ref, d_ref):
    i = pl.program_id(0)
    j = pl.program_id(1)
    a = nmh_ref[pl.ds(i * _CB, _CB), :]
    b = emh_ref[pl.ds(j * _CB, _CB), :]
    m = lax.dot_general(a, b, (((1,), (1,)), ((), ())),
                        preferred_element_type=jnp.float32)
    e = jnp.exp2(-jnp.abs(m))
    ones = jnp.ones((_CB,), jnp.float32)
    rs = lax.dot_general(e, ones, (((1,), (0,)), ((), ())),
                         preferred_element_type=jnp.float32
                         ).reshape(_CB // 128, 128)
    cs = lax.dot_general(ones, e, (((0,), (0,)), ((), ())),
                         preferred_element_type=jnp.float32
                         ).reshape(_CB // 128, 128)
    rsl = pl.ds(i * (_CB // 128), _CB // 128)
    csl = pl.ds(j * (_CB // 128), _CB // 128)

    @pl.when(j == 0)
    def _():
        s1_ref[rsl, :] = jnp.zeros((_CB // 128, 128), jnp.float32)

    @pl.when(i == 0)
    def _():
        s0_ref[csl, :] = jnp.zeros((_CB // 128, 128), jnp.float32)

    s1_ref[rsl, :] += rs
    s0_ref[csl, :] += cs

    @pl.when(i == j)
    def _():
        r = lax.broadcasted_iota(jnp.int32, (_CB, _CB), 0)
        c = lax.broadcasted_iota(jnp.int32, (_CB, _CB), 1)
        diag = jnp.sum(jnp.where(r == c, m, 0.0), axis=1)
        d_ref[rsl, :] = diag.reshape(_CB // 128, 128)

    @pl.when((i == _CG - 1) & (j == _CG - 1))
    def _():
        o_ref[...] = (jnp.abs(d_ref[...]) * (1.0 / _LOG2E) - jnp.log(2.0)
                      + jnp.log(s0_ref[...] + s1_ref[...]))


def _contrast(nmh, emh):
    out = pl.pallas_call(
        _contrast_kernel,
        grid=(_CG, _CG),
        in_specs=[pl.BlockSpec((NE, MAP), lambda i, j: (0, 0)),
                  pl.BlockSpec((NE, MAP), lambda i, j: (0, 0))],
        out_specs=pl.BlockSpec((NE // 128, 128), lambda i, j: (0, 0)),
        out_shape=jax.ShapeDtypeStruct((NE // 128, 128), jnp.float32),
        scratch_shapes=[pltpu.VMEM((NE // 128, 128), jnp.float32),
                        pltpu.VMEM((NE // 128, 128), jnp.float32),
                        pltpu.VMEM((NE // 128, 128), jnp.float32)],
    )(nmh, emh)
    return out.reshape(NE)


def kernel(nodes_feature, edges_feature, edge_index, hyperedge_index,
           gcn_w1, gcn_b1, gcn_w2, gcn_b2,
           hgc_w1, hgc_b1, hgc_w2, hgc_b2,
           node_w, node_b, edge_w, edge_b):
    ei0 = edge_index[0]
    ei1 = edge_index[1]
    hi0 = hyperedge_index[0]
    hi1 = hyperedge_index[1]

    x1e = _mm(edges_feature, hgc_w1)

    ones_tab = jnp.ones((8, 16), jnp.float32)
    cdst = jnp.concatenate([hi0, NE + hi1, 2 * NE + ei1])
    cnt, t1 = _sc_dual(
        dict(table=ones_tab, src=jnp.zeros_like(cdst), dst=cdst,
             n_out=_CNT_ROWS, const_rows=True),
        dict(table=x1e, src=hi0, dst=hi1, n_out=NH))
    dblk = 0
    bblk = NE // _RB
    gblk = 2 * NE // _RB

    z1 = _mm_dinv(nodes_feature, gcn_w1, cnt, gblk)

    z1h = z1.reshape(2 * NN, MAP)
    s1l, s1r = _sc_dual(
        dict(table=z1h, src=2 * ei0, dst=ei1, n_out=NN),
        dict(table=z1h, src=2 * ei0 + 1, dst=ei1, n_out=NN))

    z2, he1 = _gcn_layer2_z(s1l, s1r, z1, gcn_b1, gcn_w2, cnt, gblk,
                            t1, bblk)

    z2h = z2.reshape(2 * NN, MAP)
    u1, s2l = _sc_dual(
        dict(table=he1, src=hi1, dst=hi0, n_out=NE, chunk=32),
        dict(table=z2h, src=2 * ei0, dst=ei1, n_out=NN))

    x2e = _hyper_layer_out(u1, hgc_b1, hgc_w2, cnt, dblk)

    t2, s2r = _sc_dual(
        dict(table=x2e, src=hi0, dst=hi1, n_out=NH, chunk=32),
        dict(table=z2h, src=2 * ei0 + 1, dst=ei1, n_out=NN))

    pq, he2 = _pq(s2l, s2r, z2, gcn_b2, node_w[:F], node_w[F:], cnt, gblk,
                  t2, bblk)

    iota_e = jnp.arange(NE, dtype=jnp.int32)
    nmsrc = jnp.concatenate([ei0, NN + ei1])
    nmdst = jnp.concatenate([iota_e, iota_e])
    u2, nmp = _sc_dual(
        dict(table=he2, src=hi1, dst=hi0, n_out=NE),
        dict(table=pq.reshape(2 * NN, MAP), src=nmsrc, dst=nmdst, n_out=NE))

    emh, nmh = _maps(u2, hgc_b2, edge_w, edge_b, nmp, node_b, cnt, dblk)
    return _contrast(nmh, emh)


# speedup vs baseline: 1.2018x; 1.0362x over previous
import functools

import jax
import jax.numpy as jnp
from jax import lax
from jax.experimental import pallas as pl
from jax.experimental.pallas import tpu as pltpu
from jax.experimental.pallas import tpu_sc as plsc

NN = 10000
NE = 8192
NH = 8192
NNZ = 32768
F = 128
MAP = 64

_NS = 16
_CHUNK = 128
_CNT_ROWS = 2 * NE + NN
_LOG2E = 1.4426950408889634


def _pad128(n):
    return -(-n // 128) * 128


def _emit_op(op, sid, tab_h, src_h, dst_h, zeros_h, out_h, sbuf, dbuf,
             rowss, gsems, ssems, acc):
    c = op["chunk"]
    nchunk = (op["k"] // _NS) // c
    const = bool(op.get("const_rows"))
    n_pad = op["n_pad"]
    rpt = n_pad // _NS
    zoff = 0
    while zoff < rpt:
        zn = min(_CHUNK, rpt - zoff)
        pltpu.sync_copy(zeros_h.at[pl.ds(0, zn)],
                        acc.at[pl.ds(sid * rpt + zoff, zn)])
        zoff += zn
    pltpu.sync_copy(dst_h.at[pl.ds(sid * nchunk, nchunk)], dbuf)
    if const:
        pltpu.sync_copy(src_h.at[pl.ds(sid * nchunk, 8)], sbuf)
        pltpu.async_copy(tab_h.at[sbuf.at[0]], rowss[0], gsems[0]).wait()
    else:
        pltpu.sync_copy(src_h.at[pl.ds(sid * nchunk, nchunk)], sbuf)
    plsc.subcore_barrier()
    gd = {}
    sd = {}
    if not const:
        gd[0] = pltpu.async_copy(tab_h.at[sbuf.at[0]], rowss[0], gsems[0])
    for t in range(nchunk):
        b = t % 2
        if t >= 1:
            sd[t - 1].wait()
        if not const and t + 1 < nchunk:
            gd[t + 1] = pltpu.async_copy(tab_h.at[sbuf.at[t + 1]],
                                         rowss[1 - b], gsems[1 - b])
        if not const:
            gd[t].wait()
        sd[t] = pltpu.async_copy(rowss[b] if not const else rowss[0],
                                 acc.at[dbuf.at[t]], ssems[b], add=True)
    sd[nchunk - 1].wait()
    plsc.subcore_barrier()
    pltpu.sync_copy(acc.at[pl.ds(sid * rpt, rpt)],
                    out_h.at[pl.ds(sid * rpt, rpt)])


def _sc_dual(op_a, op_b):
    ops = []
    for op in (op_a, op_b):
        o = dict(op)
        (o["k"],) = o["src"].shape
        o["w"] = o["table"].shape[1]
        o["n_pad"] = _pad128(o["n_out"])
        npt = o["k"] // _NS
        if "chunk" not in o:
            o["chunk"] = next(c for c in (64, 32)
                              if npt % c == 0 and (npt // c) % 8 == 0)
        o["src"] = o["src"].reshape(-1, o["chunk"])
        o["dst"] = o["dst"].reshape(-1, o["chunk"])
        o["zeros"] = jnp.zeros((_CHUNK, o["w"]), jnp.float32)
        ops.append(o)
    a, b = ops
    mesh = plsc.VectorSubcoreMesh(core_axis_name="c", subcore_axis_name="s")

    def body(ta, sa, da, za, tb, sb, db, zb, oa, ob,
             sbuf_a, dbuf_a, rows_a0, rows_a1,
             sbuf_b, dbuf_b, rows_b0, rows_b1,
             acc_a, acc_b, g0a, g1a, s0a, s1a, g0b, g1b, s0b, s1b):
        cid = lax.axis_index("c")
        sid = lax.axis_index("s")

        @pl.when(cid == 0)
        def _():
            _emit_op(a, sid, ta, sa, da, za, oa, sbuf_a, dbuf_a,
                     [rows_a0, rows_a1], [g0a, g1a], [s0a, s1a], acc_a)

        @pl.when(cid == 1)
        def _():
            _emit_op(b, sid, tb, sb, db, zb, ob, sbuf_b, dbuf_b,
                     [rows_b0, rows_b1], [g0b, g1b], [s0b, s1b], acc_b)

    f = pl.kernel(
        body,
        out_type=(jax.ShapeDtypeStruct((a["n_pad"], a["w"]), jnp.float32),
                  jax.ShapeDtypeStruct((b["n_pad"], b["w"]), jnp.float32)),
        mesh=mesh,
        compiler_params=pltpu.CompilerParams(use_tc_tiling_on_sc=False),
        scratch_types=[
            pltpu.VMEM((8 if a.get("const_rows") else a["k"] // _NS // a["chunk"],
                        a["chunk"]), jnp.int32),
            pltpu.VMEM((a["k"] // _NS // a["chunk"], a["chunk"]), jnp.int32),
            pltpu.VMEM((a["chunk"], a["w"]), jnp.float32),
            pltpu.VMEM((a["chunk"], a["w"]), jnp.float32),
            pltpu.VMEM((8 if b.get("const_rows") else b["k"] // _NS // b["chunk"],
                        b["chunk"]), jnp.int32),
            pltpu.VMEM((b["k"] // _NS // b["chunk"], b["chunk"]), jnp.int32),
            pltpu.VMEM((b["chunk"], b["w"]), jnp.float32),
            pltpu.VMEM((b["chunk"], b["w"]), jnp.float32),
            pltpu.VMEM_SHARED((a["n_pad"], a["w"]), jnp.float32),
            pltpu.VMEM_SHARED((b["n_pad"], b["w"]), jnp.float32),
        ] + [pltpu.SemaphoreType.DMA] * 8,
    )
    return f(a["table"], a["src"], a["dst"], a["zeros"],
             b["table"], b["src"], b["dst"], b["zeros"])


_RB = 512


def _mm_kernel(x_ref, w_ref, o_ref):
    o_ref[...] = jnp.dot(x_ref[...], w_ref[...], preferred_element_type=jnp.float32)


def _mm(x, w):
    n = x.shape[0]
    g = pl.cdiv(n, _RB)
    return pl.pallas_call(
        _mm_kernel,
        grid=(g,),
        in_specs=[pl.BlockSpec((_RB, x.shape[1]), lambda i: (i, 0)),
                  pl.BlockSpec(w.shape, lambda i: (0, 0))],
        out_specs=pl.BlockSpec((_RB, w.shape[1]), lambda i: (i, 0)),
        out_shape=jax.ShapeDtypeStruct((n, w.shape[1]), jnp.float32),
    )(x, w)


def _mm_dinv_kernel(x_ref, w_ref, cnt_ref, o_ref):
    dinv = lax.rsqrt(1.0 + cnt_ref[:, 0:1])
    o_ref[...] = jnp.dot(x_ref[...], w_ref[...],
                         preferred_element_type=jnp.float32) * dinv


def _mm_dinv(x, w, cnt, cblk0):
    n = x.shape[0]
    g = pl.cdiv(n, _RB)
    return pl.pallas_call(
        _mm_dinv_kernel,
        grid=(g,),
        in_specs=[pl.BlockSpec((_RB, x.shape[1]), lambda i: (i, 0)),
                  pl.BlockSpec(w.shape, lambda i: (0, 0)),
                  pl.BlockSpec((_RB, 16), lambda i: (cblk0 + i, 0))],
        out_specs=pl.BlockSpec((_RB, w.shape[1]), lambda i: (i, 0)),
        out_shape=jax.ShapeDtypeStruct((n, w.shape[1]), jnp.float32),
    )(x, w, cnt)


def _edge_clamp(i):
    return jnp.minimum(i, NE // _RB - 1)


def _leaky(x):
    return jnp.where(x >= 0, x, 0.01 * x)


def _gcn2_kernel(sl_ref, sr_ref, z_ref, b_ref, w_ref, cnt_ref,
                 t_ref, cntb_ref, o_ref, he_ref):
    dinv = lax.rsqrt(1.0 + cnt_ref[:, 0:1])
    s = jnp.concatenate([sl_ref[...], sr_ref[...]], axis=1)
    h = _leaky(dinv * (s + z_ref[...]) + b_ref[...])
    o_ref[...] = jnp.dot(h, w_ref[...], preferred_element_type=jnp.float32) * dinv
    c = cntb_ref[:, 0:1]
    he_ref[...] = jnp.where(c > 0, 1.0 / c, 0.0) * t_ref[...]


def _gcn_layer2_z(sl, sr, z, b, w, cnt, cblk0, t, tblk0):
    n = z.shape[0]
    g = pl.cdiv(n, _RB)
    return pl.pallas_call(
        _gcn2_kernel,
        grid=(g,),
        in_specs=[pl.BlockSpec((_RB, MAP), lambda i: (i, 0)),
                  pl.BlockSpec((_RB, MAP), lambda i: (i, 0)),
                  pl.BlockSpec((_RB, F), lambda i: (i, 0)),
                  pl.BlockSpec((1, F), lambda i: (0, 0)),
                  pl.BlockSpec((F, F), lambda i: (0, 0)),
                  pl.BlockSpec((_RB, 16), lambda i: (cblk0 + i, 0)),
                  pl.BlockSpec((_RB, F), lambda i: (_edge_clamp(i), 0)),
                  pl.BlockSpec((_RB, 16),
                               lambda i: (tblk0 + _edge_clamp(i), 0))],
        out_specs=[pl.BlockSpec((_RB, F), lambda i: (i, 0)),
                   pl.BlockSpec((_RB, F), lambda i: (_edge_clamp(i), 0))],
        out_shape=[jax.ShapeDtypeStruct((n, F), jnp.float32),
                   jax.ShapeDtypeStruct((NE, F), jnp.float32)],
    )(sl, sr, z, b.reshape(1, F), w, cnt, t, cnt)


def _hyper_out_kernel(u_ref, b_ref, w_ref, cnt_ref, o_ref):
    c = cnt_ref[:, 0:1]
    dinv = jnp.where(c > 0, 1.0 / c, 0.0)
    g = _leaky(dinv * u_ref[...] + b_ref[...])
    o_ref[...] = jnp.dot(g, w_ref[...], preferred_element_type=jnp.float32)


def _hyper_layer_out(u, b, w, cnt, cblk0):
    g = NE // _RB
    return pl.pallas_call(
        _hyper_out_kernel,
        grid=(g,),
        in_specs=[pl.BlockSpec((_RB, F), lambda i: (i, 0)),
                  pl.BlockSpec((1, F), lambda i: (0, 0)),
                  pl.BlockSpec(w.shape, lambda i: (0, 0)),
                  pl.BlockSpec((_RB, 16), lambda i: (cblk0 + i, 0))],
        out_specs=pl.BlockSpec((_RB, w.shape[1]), lambda i: (i, 0)),
        out_shape=jax.ShapeDtypeStruct((NE, w.shape[1]), jnp.float32),
    )(u, b.reshape(1, F), w, cnt)


def _pq_kernel(sl_ref, sr_ref, z_ref, b_ref, wt_ref, wb_ref, cnt_ref,
               t_ref, cntb_ref, o_ref, he_ref):
    dinv = lax.rsqrt(1.0 + cnt_ref[:, 0:1])
    s = jnp.concatenate([sl_ref[...], sr_ref[...]], axis=1)
    h2 = _leaky(dinv * (s + z_ref[...]) + b_ref[...])
    p = jnp.dot(h2, wt_ref[...], preferred_element_type=jnp.float32)
    q = jnp.dot(h2, wb_ref[...], preferred_element_type=jnp.float32)
    o_ref[...] = jnp.stack([p, q])
    c = cntb_ref[:, 0:1]
    he_ref[...] = jnp.where(c > 0, 1.0 / c, 0.0) * t_ref[...]


def _pq(sl, sr, z2, b2, wtop, wbot, cnt, cblk0, t, tblk0):
    n = z2.shape[0]
    g = pl.cdiv(n, _RB)
    return pl.pallas_call(
        _pq_kernel,
        grid=(g,),
        in_specs=[pl.BlockSpec((_RB, MAP), lambda i: (i, 0)),
                  pl.BlockSpec((_RB, MAP), lambda i: (i, 0)),
                  pl.BlockSpec((_RB, F), lambda i: (i, 0)),
                  pl.BlockSpec((1, F), lambda i: (0, 0)),
                  pl.BlockSpec((F, MAP), lambda i: (0, 0)),
                  pl.BlockSpec((F, MAP), lambda i: (0, 0)),
                  pl.BlockSpec((_RB, 16), lambda i: (cblk0 + i, 0)),
                  pl.BlockSpec((_RB, F), lambda i: (_edge_clamp(i), 0)),
                  pl.BlockSpec((_RB, 16),
                               lambda i: (tblk0 + _edge_clamp(i), 0))],
        out_specs=[pl.BlockSpec((2, _RB, MAP), lambda i: (0, i, 0)),
                   pl.BlockSpec((_RB, F), lambda i: (_edge_clamp(i), 0))],
        out_shape=[jax.ShapeDtypeStruct((2, n, MAP), jnp.float32),
                   jax.ShapeDtypeStruct((NE, F), jnp.float32)],
    )(sl, sr, z2, b2.reshape(1, F), wtop, wbot, cnt, t, cnt)


def _maps_kernel(u_ref, hb_ref, ew_ref, eb_ref, nm_ref, nb_ref, cnt_ref,
                 emh_ref, nmh_ref):
    c = cnt_ref[:, 0:1]
    dinv = jnp.where(c > 0, 1.0 / c, 0.0)
    g2 = _leaky(dinv * u_ref[...] + hb_ref[...])
    em = jnp.dot(g2, ew_ref[...], preferred_element_type=jnp.float32) + eb_ref[...]
    emh = em * (lax.rsqrt(jnp.sum(em * em, axis=1, keepdims=True)) * _LOG2E)
    emh_ref[...] = emh.astype(jnp.bfloat16)
    nm = nm_ref[...] + nb_ref[...]
    nmh = nm * lax.rsqrt(jnp.sum(nm * nm, axis=1, keepdims=True))
    nmh_ref[...] = nmh.astype(jnp.bfloat16)


def _maps(u2, hgc_b2, edge_w, edge_b, nmp, node_b, cnt, cblk0):
    g = NE // _RB
    return pl.pallas_call(
        _maps_kernel,
        grid=(g,),
        in_specs=[pl.BlockSpec((_RB, F), lambda i: (i, 0)),
                  pl.BlockSpec((1, F), lambda i: (0, 0)),
                  pl.BlockSpec((F, MAP), lambda i: (0, 0)),
                  pl.BlockSpec((1, MAP), lambda i: (0, 0)),
                  pl.BlockSpec((_RB, MAP), lambda i: (i, 0)),
                  pl.BlockSpec((1, MAP), lambda i: (0, 0)),
                  pl.BlockSpec((_RB, 16), lambda i: (cblk0 + i, 0))],
        out_specs=[pl.BlockSpec((_RB, MAP), lambda i: (i, 0)),
                   pl.BlockSpec((_RB, MAP), lambda i: (i, 0))],
        out_shape=[jax.ShapeDtypeStruct((NE, MAP), jnp.bfloat16),
                   jax.ShapeDtypeStruct((NE, MAP), jnp.bfloat16)],
    )(u2, hgc_b2.reshape(1, F), edge_w, edge_b.reshape(1, MAP),
      nmp, node_b.reshape(1, MAP), cnt)


_CB = 1024
_CG = NE // _CB


def _contrast_kernel(nmh_ref, emh_ref, o_ref, s0_ref, s1_ref, d_ref):
    i = pl.program_id(0)
    j = pl.program_id(1)
    a = nmh_ref[pl.ds(i * _CB, _CB), :]
    b = emh_ref[pl.ds(j * _CB, _CB), :]
    m = lax.dot_general(a, b, (((1,), (1,)), ((), ())),
                        preferred_element_type=jnp.float32)
    e = jnp.exp2(-jnp.abs(m))
    ones = jnp.ones((_CB,), jnp.float32)
    rs = lax.dot_general(e, ones, (((1,), (0,)), ((), ())),
                         preferred_element_type=jnp.float32
                         ).reshape(_CB // 128, 128)
    cs = lax.dot_general(ones, e, (((0,), (0,)), ((), ())),
                         preferred_element_type=jnp.float32
                         ).reshape(_CB // 128, 128)
    rsl = pl.ds(i * (_CB // 128), _CB // 128)
    csl = pl.ds(j * (_CB // 128), _CB // 128)

    @pl.when(j == 0)
    def _():
        s1_ref[rsl, :] = jnp.zeros((_CB // 128, 128), jnp.float32)

    @pl.when(i == 0)
    def _():
        s0_ref[csl, :] = jnp.zeros((_CB // 128, 128), jnp.float32)

    s1_ref[rsl, :] += rs
    s0_ref[csl, :] += cs

    @pl.when(i == j)
    def _():
        r = lax.broadcasted_iota(jnp.int32, (_CB, _CB), 0)
        c = lax.broadcasted_iota(jnp.int32, (_CB, _CB), 1)
        diag = jnp.sum(jnp.where(r == c, m, 0.0), axis=1)
        d_ref[rsl, :] = diag.reshape(_CB // 128, 128)

    @pl.when((i == _CG - 1) & (j == _CG - 1))
    def _():
        o_ref[...] = (jnp.abs(d_ref[...]) * (1.0 / _LOG2E) - jnp.log(2.0)
                      + jnp.log(s0_ref[...] + s1_ref[...]))


def _contrast(nmh, emh):
    out = pl.pallas_call(
        _contrast_kernel,
        grid=(_CG, _CG),
        in_specs=[pl.BlockSpec((NE, MAP), lambda i, j: (0, 0)),
                  pl.BlockSpec((NE, MAP), lambda i, j: (0, 0))],
        out_specs=pl.BlockSpec((NE // 128, 128), lambda i, j: (0, 0)),
        out_shape=jax.ShapeDtypeStruct((NE // 128, 128), jnp.float32),
        scratch_shapes=[pltpu.VMEM((NE // 128, 128), jnp.float32),
                        pltpu.VMEM((NE // 128, 128), jnp.float32),
                        pltpu.VMEM((NE // 128, 128), jnp.float32)],
    )(nmh, emh)
    return out.reshape(NE)


def kernel(nodes_feature, edges_feature, edge_index, hyperedge_index,
           gcn_w1, gcn_b1, gcn_w2, gcn_b2,
           hgc_w1, hgc_b1, hgc_w2, hgc_b2,
           node_w, node_b, edge_w, edge_b):
    ei0 = edge_index[0]
    ei1 = edge_index[1]
    hi0 = hyperedge_index[0]
    hi1 = hyperedge_index[1]

    x1e = _mm(edges_feature, hgc_w1)

    ones_tab = jnp.ones((8, 16), jnp.float32)
    npad = 81920 - (2 * NNZ + NE)
    dump = _CNT_ROWS + (jnp.arange(npad, dtype=jnp.int32)
                        % (_pad128(_CNT_ROWS) - _CNT_ROWS))
    cdst = jnp.concatenate([hi0, NE + hi1, 2 * NE + ei1, dump])
    cnt, t1 = _sc_dual(
        dict(table=ones_tab, src=jnp.zeros_like(cdst), dst=cdst,
             n_out=_CNT_ROWS, const_rows=True, chunk=128),
        dict(table=x1e, src=hi0, dst=hi1, n_out=NH))
    dblk = 0
    bblk = NE // _RB
    gblk = 2 * NE // _RB

    z1 = _mm_dinv(nodes_feature, gcn_w1, cnt, gblk)

    z1h = z1.reshape(2 * NN, MAP)
    s1l, s1r = _sc_dual(
        dict(table=z1h, src=2 * ei0, dst=ei1, n_out=NN),
        dict(table=z1h, src=2 * ei0 + 1, dst=ei1, n_out=NN))

    z2, he1 = _gcn_layer2_z(s1l, s1r, z1, gcn_b1, gcn_w2, cnt, gblk,
                            t1, bblk)

    z2h = z2.reshape(2 * NN, MAP)
    u1, s2l = _sc_dual(
        dict(table=he1, src=hi1, dst=hi0, n_out=NE, chunk=64),
        dict(table=z2h, src=2 * ei0, dst=ei1, n_out=NN, chunk=16))

    x2e = _hyper_layer_out(u1, hgc_b1, hgc_w2, cnt, dblk)

    t2, s2r = _sc_dual(
        dict(table=x2e, src=hi0, dst=hi1, n_out=NH, chunk=64),
        dict(table=z2h, src=2 * ei0 + 1, dst=ei1, n_out=NN, chunk=16))

    pq, he2 = _pq(s2l, s2r, z2, gcn_b2, node_w[:F], node_w[F:], cnt, gblk,
                  t2, bblk)

    iota_e = jnp.arange(NE, dtype=jnp.int32)
    nmsrc = jnp.concatenate([ei0, NN + ei1])
    nmdst = jnp.concatenate([iota_e, iota_e])
    u2, nmp = _sc_dual(
        dict(table=he2, src=hi1, dst=hi0, n_out=NE),
        dict(table=pq.reshape(2 * NN, MAP), src=nmsrc, dst=nmdst, n_out=NE))

    emh, nmh = _maps(u2, hgc_b2, edge_w, edge_b, nmp, node_b, cnt, dblk)
    return _contrast(nmh, emh)


# speedup vs baseline: 1.2349x; 1.0275x over previous
import functools

import jax
import jax.numpy as jnp
from jax import lax
from jax.experimental import pallas as pl
from jax.experimental.pallas import tpu as pltpu
from jax.experimental.pallas import tpu_sc as plsc

NN = 10000
NE = 8192
NH = 8192
NNZ = 32768
F = 128
MAP = 64

_NS = 16
_CHUNK = 128
_CNT_ROWS = 2 * NE + NN
_LOG2E = 1.4426950408889634


def _pad128(n):
    return -(-n // 128) * 128


def _emit_op(op, sid, tab_h, src_h, dst_h, zeros_h, out_h, sbuf, dbuf,
             rowss, gsems, ssems, acc):
    c = op["chunk"]
    nchunk = (op["k"] // _NS) // c
    const = bool(op.get("const_rows"))
    n_pad = op["n_pad"]
    rpt = n_pad // _NS
    zoff = 0
    while zoff < rpt:
        zn = min(_CHUNK, rpt - zoff)
        pltpu.sync_copy(zeros_h.at[pl.ds(0, zn)],
                        acc.at[pl.ds(sid * rpt + zoff, zn)])
        zoff += zn
    pltpu.sync_copy(dst_h.at[pl.ds(sid * nchunk, nchunk)], dbuf)
    if const:
        pltpu.sync_copy(src_h.at[pl.ds(sid * nchunk, 8)], sbuf)
        pltpu.async_copy(tab_h.at[sbuf.at[0]], rowss[0], gsems[0]).wait()
    else:
        pltpu.sync_copy(src_h.at[pl.ds(sid * nchunk, nchunk)], sbuf)
    plsc.subcore_barrier()
    gd = {}
    sd = {}
    if not const:
        gd[0] = pltpu.async_copy(tab_h.at[sbuf.at[0]], rowss[0], gsems[0])
    for t in range(nchunk):
        b = t % 2
        if t >= 1:
            sd[t - 1].wait()
        if not const and t + 1 < nchunk:
            gd[t + 1] = pltpu.async_copy(tab_h.at[sbuf.at[t + 1]],
                                         rowss[1 - b], gsems[1 - b])
        if not const:
            gd[t].wait()
        sd[t] = pltpu.async_copy(rowss[b] if not const else rowss[0],
                                 acc.at[dbuf.at[t]], ssems[b], add=True)
    sd[nchunk - 1].wait()
    plsc.subcore_barrier()
    pltpu.sync_copy(acc.at[pl.ds(sid * rpt, rpt)],
                    out_h.at[pl.ds(sid * rpt, rpt)])


def _sc_dual(op_a, op_b):
    ops = []
    for op in (op_a, op_b):
        o = dict(op)
        (o["k"],) = o["src"].shape
        o["w"] = o["table"].shape[1]
        o["n_pad"] = _pad128(o["n_out"])
        npt = o["k"] // _NS
        if "chunk" not in o:
            o["chunk"] = next(c for c in (64, 32)
                              if npt % c == 0 and (npt // c) % 8 == 0)
        o["src"] = o["src"].reshape(-1, o["chunk"])
        o["dst"] = o["dst"].reshape(-1, o["chunk"])
        o["zeros"] = jnp.zeros((_CHUNK, o["w"]), jnp.float32)
        ops.append(o)
    a, b = ops
    mesh = plsc.VectorSubcoreMesh(core_axis_name="c", subcore_axis_name="s")

    def body(ta, sa, da, za, tb, sb, db, zb, oa, ob,
             sbuf_a, dbuf_a, rows_a0, rows_a1,
             sbuf_b, dbuf_b, rows_b0, rows_b1,
             acc_a, acc_b, g0a, g1a, s0a, s1a, g0b, g1b, s0b, s1b):
        cid = lax.axis_index("c")
        sid = lax.axis_index("s")

        @pl.when(cid == 0)
        def _():
            _emit_op(a, sid, ta, sa, da, za, oa, sbuf_a, dbuf_a,
                     [rows_a0, rows_a1], [g0a, g1a], [s0a, s1a], acc_a)

        @pl.when(cid == 1)
        def _():
            _emit_op(b, sid, tb, sb, db, zb, ob, sbuf_b, dbuf_b,
                     [rows_b0, rows_b1], [g0b, g1b], [s0b, s1b], acc_b)

    f = pl.kernel(
        body,
        out_type=(jax.ShapeDtypeStruct((a["n_pad"], a["w"]), jnp.float32),
                  jax.ShapeDtypeStruct((b["n_pad"], b["w"]), jnp.float32)),
        mesh=mesh,
        compiler_params=pltpu.CompilerParams(use_tc_tiling_on_sc=False),
        scratch_types=[
            pltpu.VMEM((8 if a.get("const_rows") else a["k"] // _NS // a["chunk"],
                        a["chunk"]), jnp.int32),
            pltpu.VMEM((a["k"] // _NS // a["chunk"], a["chunk"]), jnp.int32),
            pltpu.VMEM((a["chunk"], a["w"]), jnp.float32),
            pltpu.VMEM((a["chunk"], a["w"]), jnp.float32),
            pltpu.VMEM((8 if b.get("const_rows") else b["k"] // _NS // b["chunk"],
                        b["chunk"]), jnp.int32),
            pltpu.VMEM((b["k"] // _NS // b["chunk"], b["chunk"]), jnp.int32),
            pltpu.VMEM((b["chunk"], b["w"]), jnp.float32),
            pltpu.VMEM((b["chunk"], b["w"]), jnp.float32),
            pltpu.VMEM_SHARED((a["n_pad"], a["w"]), jnp.float32),
            pltpu.VMEM_SHARED((b["n_pad"], b["w"]), jnp.float32),
        ] + [pltpu.SemaphoreType.DMA] * 8,
    )
    return f(a["table"], a["src"], a["dst"], a["zeros"],
             b["table"], b["src"], b["dst"], b["zeros"])


_RB = 512


def _mm_kernel(x_ref, w_ref, o_ref):
    o_ref[...] = jnp.dot(x_ref[...], w_ref[...], preferred_element_type=jnp.float32)


def _mm(x, w):
    n = x.shape[0]
    g = pl.cdiv(n, _RB)
    return pl.pallas_call(
        _mm_kernel,
        grid=(g,),
        in_specs=[pl.BlockSpec((_RB, x.shape[1]), lambda i: (i, 0)),
                  pl.BlockSpec(w.shape, lambda i: (0, 0))],
        out_specs=pl.BlockSpec((_RB, w.shape[1]), lambda i: (i, 0)),
        out_shape=jax.ShapeDtypeStruct((n, w.shape[1]), jnp.float32),
    )(x, w)


def _mm_dinv_kernel(x_ref, w_ref, cnt_ref, o_ref):
    dinv = lax.rsqrt(1.0 + cnt_ref[:, 0:1])
    o_ref[...] = jnp.dot(x_ref[...], w_ref[...],
                         preferred_element_type=jnp.float32) * dinv


def _mm_dinv(x, w, cnt, cblk0):
    n = x.shape[0]
    g = pl.cdiv(n, _RB)
    return pl.pallas_call(
        _mm_dinv_kernel,
        grid=(g,),
        in_specs=[pl.BlockSpec((_RB, x.shape[1]), lambda i: (i, 0)),
                  pl.BlockSpec(w.shape, lambda i: (0, 0)),
                  pl.BlockSpec((_RB, 16), lambda i: (cblk0 + i, 0))],
        out_specs=pl.BlockSpec((_RB, w.shape[1]), lambda i: (i, 0)),
        out_shape=jax.ShapeDtypeStruct((n, w.shape[1]), jnp.float32),
    )(x, w, cnt)


def _edge_clamp(i):
    return jnp.minimum(i, NE // _RB - 1)


def _leaky(x):
    return jnp.where(x >= 0, x, 0.01 * x)


def _gcn2_kernel(sl_ref, sr_ref, z_ref, b_ref, w_ref, cnt_ref,
                 t_ref, cntb_ref, o_ref, he_ref):
    dinv = lax.rsqrt(1.0 + cnt_ref[:, 0:1])
    s = jnp.concatenate([sl_ref[...], sr_ref[...]], axis=1)
    h = _leaky(dinv * (s + z_ref[...]) + b_ref[...])
    o_ref[...] = jnp.dot(h, w_ref[...], preferred_element_type=jnp.float32) * dinv
    c = cntb_ref[:, 0:1]
    he_ref[...] = jnp.where(c > 0, 1.0 / c, 0.0) * t_ref[...]


def _gcn_layer2_z(sl, sr, z, b, w, cnt, cblk0, t, tblk0):
    n = z.shape[0]
    g = pl.cdiv(n, _RB)
    return pl.pallas_call(
        _gcn2_kernel,
        grid=(g,),
        in_specs=[pl.BlockSpec((_RB, MAP), lambda i: (i, 0)),
                  pl.BlockSpec((_RB, MAP), lambda i: (i, 0)),
                  pl.BlockSpec((_RB, F), lambda i: (i, 0)),
                  pl.BlockSpec((1, F), lambda i: (0, 0)),
                  pl.BlockSpec((F, F), lambda i: (0, 0)),
                  pl.BlockSpec((_RB, 16), lambda i: (cblk0 + i, 0)),
                  pl.BlockSpec((_RB, F), lambda i: (_edge_clamp(i), 0)),
                  pl.BlockSpec((_RB, 16),
                               lambda i: (tblk0 + _edge_clamp(i), 0))],
        out_specs=[pl.BlockSpec((_RB, F), lambda i: (i, 0)),
                   pl.BlockSpec((_RB, F), lambda i: (_edge_clamp(i), 0))],
        out_shape=[jax.ShapeDtypeStruct((n, F), jnp.float32),
                   jax.ShapeDtypeStruct((NE, F), jnp.float32)],
    )(sl, sr, z, b.reshape(1, F), w, cnt, t, cnt)


def _hyper_out_kernel(u_ref, b_ref, w_ref, cnt_ref, o_ref):
    c = cnt_ref[:, 0:1]
    dinv = jnp.where(c > 0, 1.0 / c, 0.0)
    g = _leaky(dinv * u_ref[...] + b_ref[...])
    o_ref[...] = jnp.dot(g, w_ref[...], preferred_element_type=jnp.float32)


def _hyper_layer_out(u, b, w, cnt, cblk0):
    g = NE // _RB
    return pl.pallas_call(
        _hyper_out_kernel,
        grid=(g,),
        in_specs=[pl.BlockSpec((_RB, F), lambda i: (i, 0)),
                  pl.BlockSpec((1, F), lambda i: (0, 0)),
                  pl.BlockSpec(w.shape, lambda i: (0, 0)),
                  pl.BlockSpec((_RB, 16), lambda i: (cblk0 + i, 0))],
        out_specs=pl.BlockSpec((_RB, w.shape[1]), lambda i: (i, 0)),
        out_shape=jax.ShapeDtypeStruct((NE, w.shape[1]), jnp.float32),
    )(u, b.reshape(1, F), w, cnt)


def _pq_kernel(sl_ref, sr_ref, z_ref, b_ref, wt_ref, wb_ref, cnt_ref,
               t_ref, cntb_ref, o_ref, he_ref):
    dinv = lax.rsqrt(1.0 + cnt_ref[:, 0:1])
    s = jnp.concatenate([sl_ref[...], sr_ref[...]], axis=1)
    h2 = _leaky(dinv * (s + z_ref[...]) + b_ref[...])
    p = jnp.dot(h2, wt_ref[...], preferred_element_type=jnp.float32)
    q = jnp.dot(h2, wb_ref[...], preferred_element_type=jnp.float32)
    o_ref[...] = jnp.stack([p, q])
    c = cntb_ref[:, 0:1]
    he_ref[...] = jnp.where(c > 0, 1.0 / c, 0.0) * t_ref[...]


def _pq(sl, sr, z2, b2, wtop, wbot, cnt, cblk0, t, tblk0):
    n = z2.shape[0]
    g = pl.cdiv(n, _RB)
    return pl.pallas_call(
        _pq_kernel,
        grid=(g,),
        in_specs=[pl.BlockSpec((_RB, MAP), lambda i: (i, 0)),
                  pl.BlockSpec((_RB, MAP), lambda i: (i, 0)),
                  pl.BlockSpec((_RB, F), lambda i: (i, 0)),
                  pl.BlockSpec((1, F), lambda i: (0, 0)),
                  pl.BlockSpec((F, MAP), lambda i: (0, 0)),
                  pl.BlockSpec((F, MAP), lambda i: (0, 0)),
                  pl.BlockSpec((_RB, 16), lambda i: (cblk0 + i, 0)),
                  pl.BlockSpec((_RB, F), lambda i: (_edge_clamp(i), 0)),
                  pl.BlockSpec((_RB, 16),
                               lambda i: (tblk0 + _edge_clamp(i), 0))],
        out_specs=[pl.BlockSpec((2, _RB, MAP), lambda i: (0, i, 0)),
                   pl.BlockSpec((_RB, F), lambda i: (_edge_clamp(i), 0))],
        out_shape=[jax.ShapeDtypeStruct((2, n, MAP), jnp.float32),
                   jax.ShapeDtypeStruct((NE, F), jnp.float32)],
    )(sl, sr, z2, b2.reshape(1, F), wtop, wbot, cnt, t, cnt)


def _maps_kernel(u_ref, hb_ref, ew_ref, eb_ref, nm_ref, nb_ref, cnt_ref,
                 emh_ref, nmh_ref):
    c = cnt_ref[:, 0:1]
    dinv = jnp.where(c > 0, 1.0 / c, 0.0)
    g2 = _leaky(dinv * u_ref[...] + hb_ref[...])
    em = jnp.dot(g2, ew_ref[...], preferred_element_type=jnp.float32) + eb_ref[...]
    emh = em * (lax.rsqrt(jnp.sum(em * em, axis=1, keepdims=True)) * _LOG2E)
    emh_ref[...] = emh.astype(jnp.bfloat16)
    nm = nm_ref[...] + nb_ref[...]
    nmh = nm * lax.rsqrt(jnp.sum(nm * nm, axis=1, keepdims=True))
    nmh_ref[...] = nmh.astype(jnp.bfloat16)


def _maps(u2, hgc_b2, edge_w, edge_b, nmp, node_b, cnt, cblk0):
    g = NE // _RB
    return pl.pallas_call(
        _maps_kernel,
        grid=(g,),
        in_specs=[pl.BlockSpec((_RB, F), lambda i: (i, 0)),
                  pl.BlockSpec((1, F), lambda i: (0, 0)),
                  pl.BlockSpec((F, MAP), lambda i: (0, 0)),
                  pl.BlockSpec((1, MAP), lambda i: (0, 0)),
                  pl.BlockSpec((_RB, MAP), lambda i: (i, 0)),
                  pl.BlockSpec((1, MAP), lambda i: (0, 0)),
                  pl.BlockSpec((_RB, 16), lambda i: (cblk0 + i, 0))],
        out_specs=[pl.BlockSpec((_RB, MAP), lambda i: (i, 0)),
                   pl.BlockSpec((_RB, MAP), lambda i: (i, 0))],
        out_shape=[jax.ShapeDtypeStruct((NE, MAP), jnp.bfloat16),
                   jax.ShapeDtypeStruct((NE, MAP), jnp.bfloat16)],
    )(u2, hgc_b2.reshape(1, F), edge_w, edge_b.reshape(1, MAP),
      nmp, node_b.reshape(1, MAP), cnt)


_CB = 2048
_CG = NE // _CB


def _contrast_kernel(nmh_ref, emh_ref, o_ref, s0_ref, s1_ref, d_ref):
    i = pl.program_id(0)
    j = pl.program_id(1)
    a = nmh_ref[pl.ds(i * _CB, _CB), :]
    b = emh_ref[pl.ds(j * _CB, _CB), :]
    m = lax.dot_general(a, b, (((1,), (1,)), ((), ())),
                        preferred_element_type=jnp.float32)
    e = jnp.exp2(-jnp.abs(m))
    ones = jnp.ones((_CB,), jnp.float32)
    rs = lax.dot_general(e, ones, (((1,), (0,)), ((), ())),
                         preferred_element_type=jnp.float32
                         ).reshape(_CB // 128, 128)
    cs = lax.dot_general(ones, e, (((0,), (0,)), ((), ())),
                         preferred_element_type=jnp.float32
                         ).reshape(_CB // 128, 128)
    rsl = pl.ds(i * (_CB // 128), _CB // 128)
    csl = pl.ds(j * (_CB // 128), _CB // 128)

    @pl.when(j == 0)
    def _():
        s1_ref[rsl, :] = jnp.zeros((_CB // 128, 128), jnp.float32)

    @pl.when(i == 0)
    def _():
        s0_ref[csl, :] = jnp.zeros((_CB // 128, 128), jnp.float32)

    s1_ref[rsl, :] += rs
    s0_ref[csl, :] += cs

    @pl.when(i == j)
    def _():
        r = lax.broadcasted_iota(jnp.int32, (_CB, _CB), 0)
        c = lax.broadcasted_iota(jnp.int32, (_CB, _CB), 1)
        diag = jnp.sum(jnp.where(r == c, m, 0.0), axis=1)
        d_ref[rsl, :] = diag.reshape(_CB // 128, 128)

    @pl.when((i == _CG - 1) & (j == _CG - 1))
    def _():
        o_ref[...] = (jnp.abs(d_ref[...]) * (1.0 / _LOG2E) - jnp.log(2.0)
                      + jnp.log(s0_ref[...] + s1_ref[...]))


def _contrast(nmh, emh):
    out = pl.pallas_call(
        _contrast_kernel,
        grid=(_CG, _CG),
        in_specs=[pl.BlockSpec((NE, MAP), lambda i, j: (0, 0)),
                  pl.BlockSpec((NE, MAP), lambda i, j: (0, 0))],
        out_specs=pl.BlockSpec((NE // 128, 128), lambda i, j: (0, 0)),
        out_shape=jax.ShapeDtypeStruct((NE // 128, 128), jnp.float32),
        scratch_shapes=[pltpu.VMEM((NE // 128, 128), jnp.float32),
                        pltpu.VMEM((NE // 128, 128), jnp.float32),
                        pltpu.VMEM((NE // 128, 128), jnp.float32)],
    )(nmh, emh)
    return out.reshape(NE)


def kernel(nodes_feature, edges_feature, edge_index, hyperedge_index,
           gcn_w1, gcn_b1, gcn_w2, gcn_b2,
           hgc_w1, hgc_b1, hgc_w2, hgc_b2,
           node_w, node_b, edge_w, edge_b):
    ei0 = edge_index[0]
    ei1 = edge_index[1]
    hi0 = hyperedge_index[0]
    hi1 = hyperedge_index[1]

    x1e = _mm(edges_feature, hgc_w1)

    ones_tab = jnp.ones((8, 16), jnp.float32)
    npad = 81920 - (2 * NNZ + NE)
    dump = _CNT_ROWS + (jnp.arange(npad, dtype=jnp.int32)
                        % (_pad128(_CNT_ROWS) - _CNT_ROWS))
    cdst = jnp.concatenate([hi0, NE + hi1, 2 * NE + ei1, dump])
    cnt, t1 = _sc_dual(
        dict(table=ones_tab, src=jnp.zeros_like(cdst), dst=cdst,
             n_out=_CNT_ROWS, const_rows=True, chunk=128),
        dict(table=x1e, src=hi0, dst=hi1, n_out=NH))
    dblk = 0
    bblk = NE // _RB
    gblk = 2 * NE // _RB

    z1 = _mm_dinv(nodes_feature, gcn_w1, cnt, gblk)

    z1h = z1.reshape(2 * NN, MAP)
    s1l, s1r = _sc_dual(
        dict(table=z1h, src=2 * ei0, dst=ei1, n_out=NN),
        dict(table=z1h, src=2 * ei0 + 1, dst=ei1, n_out=NN))

    z2, he1 = _gcn_layer2_z(s1l, s1r, z1, gcn_b1, gcn_w2, cnt, gblk,
                            t1, bblk)

    z2h = z2.reshape(2 * NN, MAP)
    u1, s2l = _sc_dual(
        dict(table=he1, src=hi1, dst=hi0, n_out=NE, chunk=64),
        dict(table=z2h, src=2 * ei0, dst=ei1, n_out=NN, chunk=16))

    x2e = _hyper_layer_out(u1, hgc_b1, hgc_w2, cnt, dblk)

    t2, s2r = _sc_dual(
        dict(table=x2e, src=hi0, dst=hi1, n_out=NH, chunk=64),
        dict(table=z2h, src=2 * ei0 + 1, dst=ei1, n_out=NN, chunk=16))

    pq, he2 = _pq(s2l, s2r, z2, gcn_b2, node_w[:F], node_w[F:], cnt, gblk,
                  t2, bblk)

    iota_e = jnp.arange(NE, dtype=jnp.int32)
    nmsrc = jnp.concatenate([ei0, NN + ei1])
    nmdst = jnp.concatenate([iota_e, iota_e])
    u2, nmp = _sc_dual(
        dict(table=he2, src=hi1, dst=hi0, n_out=NE),
        dict(table=pq.reshape(2 * NN, MAP), src=nmsrc, dst=nmdst, n_out=NE))

    emh, nmh = _maps(u2, hgc_b2, edge_w, edge_b, nmp, node_b, cnt, dblk)
    return _contrast(nmh, emh)


# speedup vs baseline: 1.2578x; 1.0185x over previous
import functools

import jax
import jax.numpy as jnp
from jax import lax
from jax.experimental import pallas as pl
from jax.experimental.pallas import tpu as pltpu
from jax.experimental.pallas import tpu_sc as plsc

NN = 10000
NE = 8192
NH = 8192
NNZ = 32768
F = 128
MAP = 64

_NS = 16
_CHUNK = 128
_CNT_ROWS = 2 * NE + NN
_LOG2E = 1.4426950408889634


def _pad128(n):
    return -(-n // 128) * 128


def _emit_op(op, sid, tab_h, src_h, dst_h, zeros_h, out_h, sbuf, dbuf,
             rowss, gsems, ssems, acc):
    c = op["chunk"]
    nchunk = (op["k"] // _NS) // c
    const = bool(op.get("const_rows"))
    n_pad = op["n_pad"]
    rpt = n_pad // _NS
    zoff = 0
    while zoff < rpt:
        zn = min(_CHUNK, rpt - zoff)
        pltpu.sync_copy(zeros_h.at[pl.ds(0, zn)],
                        acc.at[pl.ds(sid * rpt + zoff, zn)])
        zoff += zn
    pltpu.sync_copy(dst_h.at[pl.ds(sid * nchunk, nchunk)], dbuf)
    if const:
        pltpu.sync_copy(src_h.at[pl.ds(sid * nchunk, 8)], sbuf)
        pltpu.async_copy(tab_h.at[sbuf.at[0]], rowss[0], gsems[0]).wait()
    else:
        pltpu.sync_copy(src_h.at[pl.ds(sid * nchunk, nchunk)], sbuf)
    plsc.subcore_barrier()
    gd = {}
    sd = {}
    if not const:
        gd[0] = pltpu.async_copy(tab_h.at[sbuf.at[0]], rowss[0], gsems[0])
    for t in range(nchunk):
        b = t % 2
        if t >= 1:
            sd[t - 1].wait()
        if not const and t + 1 < nchunk:
            gd[t + 1] = pltpu.async_copy(tab_h.at[sbuf.at[t + 1]],
                                         rowss[1 - b], gsems[1 - b])
        if not const:
            gd[t].wait()
        sd[t] = pltpu.async_copy(rowss[b] if not const else rowss[0],
                                 acc.at[dbuf.at[t]], ssems[b], add=True)
    sd[nchunk - 1].wait()
    plsc.subcore_barrier()
    pltpu.sync_copy(acc.at[pl.ds(sid * rpt, rpt)],
                    out_h.at[pl.ds(sid * rpt, rpt)])


def _sc_dual(op_a, op_b):
    ops = []
    for op in (op_a, op_b):
        o = dict(op)
        (o["k"],) = o["src"].shape
        o["w"] = o["table"].shape[1]
        o["n_pad"] = _pad128(o["n_out"])
        npt = o["k"] // _NS
        if "chunk" not in o:
            o["chunk"] = next(c for c in (64, 32)
                              if npt % c == 0 and (npt // c) % 8 == 0)
        o["src"] = o["src"].reshape(-1, o["chunk"])
        o["dst"] = o["dst"].reshape(-1, o["chunk"])
        o["zeros"] = jnp.zeros((_CHUNK, o["w"]), jnp.float32)
        ops.append(o)
    a, b = ops
    mesh = plsc.VectorSubcoreMesh(core_axis_name="c", subcore_axis_name="s")

    def body(ta, sa, da, za, tb, sb, db, zb, oa, ob,
             sbuf_a, dbuf_a, rows_a0, rows_a1,
             sbuf_b, dbuf_b, rows_b0, rows_b1,
             acc_a, acc_b, g0a, g1a, s0a, s1a, g0b, g1b, s0b, s1b):
        cid = lax.axis_index("c")
        sid = lax.axis_index("s")

        @pl.when(cid == 0)
        def _():
            _emit_op(a, sid, ta, sa, da, za, oa, sbuf_a, dbuf_a,
                     [rows_a0, rows_a1], [g0a, g1a], [s0a, s1a], acc_a)

        @pl.when(cid == 1)
        def _():
            _emit_op(b, sid, tb, sb, db, zb, ob, sbuf_b, dbuf_b,
                     [rows_b0, rows_b1], [g0b, g1b], [s0b, s1b], acc_b)

    f = pl.kernel(
        body,
        out_type=(jax.ShapeDtypeStruct((a["n_pad"], a["w"]), jnp.float32),
                  jax.ShapeDtypeStruct((b["n_pad"], b["w"]), jnp.float32)),
        mesh=mesh,
        compiler_params=pltpu.CompilerParams(use_tc_tiling_on_sc=False),
        scratch_types=[
            pltpu.VMEM((8 if a.get("const_rows") else a["k"] // _NS // a["chunk"],
                        a["chunk"]), jnp.int32),
            pltpu.VMEM((a["k"] // _NS // a["chunk"], a["chunk"]), jnp.int32),
            pltpu.VMEM((a["chunk"], a["w"]), jnp.float32),
            pltpu.VMEM((a["chunk"], a["w"]), jnp.float32),
            pltpu.VMEM((8 if b.get("const_rows") else b["k"] // _NS // b["chunk"],
                        b["chunk"]), jnp.int32),
            pltpu.VMEM((b["k"] // _NS // b["chunk"], b["chunk"]), jnp.int32),
            pltpu.VMEM((b["chunk"], b["w"]), jnp.float32),
            pltpu.VMEM((b["chunk"], b["w"]), jnp.float32),
            pltpu.VMEM_SHARED((a["n_pad"], a["w"]), jnp.float32),
            pltpu.VMEM_SHARED((b["n_pad"], b["w"]), jnp.float32),
        ] + [pltpu.SemaphoreType.DMA] * 8,
    )
    return f(a["table"], a["src"], a["dst"], a["zeros"],
             b["table"], b["src"], b["dst"], b["zeros"])


_RB = 512


def _mm_kernel(x_ref, w_ref, o_ref):
    o_ref[...] = jnp.dot(x_ref[...], w_ref[...], preferred_element_type=jnp.float32)


def _mm(x, w):
    n = x.shape[0]
    g = pl.cdiv(n, _RB)
    return pl.pallas_call(
        _mm_kernel,
        grid=(g,),
        in_specs=[pl.BlockSpec((_RB, x.shape[1]), lambda i: (i, 0)),
                  pl.BlockSpec(w.shape, lambda i: (0, 0))],
        out_specs=pl.BlockSpec((_RB, w.shape[1]), lambda i: (i, 0)),
        out_shape=jax.ShapeDtypeStruct((n, w.shape[1]), jnp.float32),
    )(x, w)


def _mm_dinv_kernel(x_ref, w_ref, cnt_ref, o_ref):
    dinv = lax.rsqrt(1.0 + cnt_ref[:, 0:1])
    o_ref[...] = jnp.dot(x_ref[...], w_ref[...],
                         preferred_element_type=jnp.float32) * dinv


def _mm_dinv(x, w, cnt, cblk0):
    n = x.shape[0]
    g = pl.cdiv(n, _RB)
    return pl.pallas_call(
        _mm_dinv_kernel,
        grid=(g,),
        in_specs=[pl.BlockSpec((_RB, x.shape[1]), lambda i: (i, 0)),
                  pl.BlockSpec(w.shape, lambda i: (0, 0)),
                  pl.BlockSpec((_RB, 16), lambda i: (cblk0 + i, 0))],
        out_specs=pl.BlockSpec((_RB, w.shape[1]), lambda i: (i, 0)),
        out_shape=jax.ShapeDtypeStruct((n, w.shape[1]), jnp.float32),
    )(x, w, cnt)


def _edge_clamp(i):
    return jnp.minimum(i, NE // _RB - 1)


def _leaky(x):
    return jnp.where(x >= 0, x, 0.01 * x)


def _gcn2_kernel(sl_ref, sr_ref, z_ref, b_ref, w_ref, cnt_ref,
                 t_ref, cntb_ref, o_ref, he_ref):
    dinv = lax.rsqrt(1.0 + cnt_ref[:, 0:1])
    s = jnp.concatenate([sl_ref[...], sr_ref[...]], axis=1)
    h = _leaky(dinv * (s + z_ref[...]) + b_ref[...])
    o_ref[...] = jnp.dot(h, w_ref[...], preferred_element_type=jnp.float32) * dinv
    c = cntb_ref[:, 0:1]
    he_ref[...] = jnp.where(c > 0, 1.0 / c, 0.0) * t_ref[...]


def _gcn_layer2_z(sl, sr, z, b, w, cnt, cblk0, t, tblk0):
    n = z.shape[0]
    g = pl.cdiv(n, _RB)
    return pl.pallas_call(
        _gcn2_kernel,
        grid=(g,),
        in_specs=[pl.BlockSpec((_RB, MAP), lambda i: (i, 0)),
                  pl.BlockSpec((_RB, MAP), lambda i: (i, 0)),
                  pl.BlockSpec((_RB, F), lambda i: (i, 0)),
                  pl.BlockSpec((1, F), lambda i: (0, 0)),
                  pl.BlockSpec((F, F), lambda i: (0, 0)),
                  pl.BlockSpec((_RB, 16), lambda i: (cblk0 + i, 0)),
                  pl.BlockSpec((_RB, F), lambda i: (_edge_clamp(i), 0)),
                  pl.BlockSpec((_RB, 16),
                               lambda i: (tblk0 + _edge_clamp(i), 0))],
        out_specs=[pl.BlockSpec((_RB, F), lambda i: (i, 0)),
                   pl.BlockSpec((_RB, F), lambda i: (_edge_clamp(i), 0))],
        out_shape=[jax.ShapeDtypeStruct((n, F), jnp.float32),
                   jax.ShapeDtypeStruct((NE, F), jnp.float32)],
    )(sl, sr, z, b.reshape(1, F), w, cnt, t, cnt)


def _hyper_out_kernel(u_ref, b_ref, w_ref, cnt_ref, o_ref):
    c = cnt_ref[:, 0:1]
    dinv = jnp.where(c > 0, 1.0 / c, 0.0)
    g = _leaky(dinv * u_ref[...] + b_ref[...])
    o_ref[...] = jnp.dot(g, w_ref[...], preferred_element_type=jnp.float32)


def _hyper_layer_out(u, b, w, cnt, cblk0):
    g = NE // _RB
    return pl.pallas_call(
        _hyper_out_kernel,
        grid=(g,),
        in_specs=[pl.BlockSpec((_RB, F), lambda i: (i, 0)),
                  pl.BlockSpec((1, F), lambda i: (0, 0)),
                  pl.BlockSpec(w.shape, lambda i: (0, 0)),
                  pl.BlockSpec((_RB, 16), lambda i: (cblk0 + i, 0))],
        out_specs=pl.BlockSpec((_RB, w.shape[1]), lambda i: (i, 0)),
        out_shape=jax.ShapeDtypeStruct((NE, w.shape[1]), jnp.float32),
    )(u, b.reshape(1, F), w, cnt)


def _pq_kernel(sl_ref, sr_ref, z_ref, b_ref, wt_ref, wb_ref, cnt_ref,
               t_ref, cntb_ref, o_ref, he_ref):
    dinv = lax.rsqrt(1.0 + cnt_ref[:, 0:1])
    s = jnp.concatenate([sl_ref[...], sr_ref[...]], axis=1)
    h2 = _leaky(dinv * (s + z_ref[...]) + b_ref[...])
    p = jnp.dot(h2, wt_ref[...], preferred_element_type=jnp.float32)
    q = jnp.dot(h2, wb_ref[...], preferred_element_type=jnp.float32)
    o_ref[...] = jnp.stack([p, q])
    c = cntb_ref[:, 0:1]
    he_ref[...] = jnp.where(c > 0, 1.0 / c, 0.0) * t_ref[...]


def _pq(sl, sr, z2, b2, wtop, wbot, cnt, cblk0, t, tblk0):
    n = z2.shape[0]
    g = pl.cdiv(n, _RB)
    return pl.pallas_call(
        _pq_kernel,
        grid=(g,),
        in_specs=[pl.BlockSpec((_RB, MAP), lambda i: (i, 0)),
                  pl.BlockSpec((_RB, MAP), lambda i: (i, 0)),
                  pl.BlockSpec((_RB, F), lambda i: (i, 0)),
                  pl.BlockSpec((1, F), lambda i: (0, 0)),
                  pl.BlockSpec((F, MAP), lambda i: (0, 0)),
                  pl.BlockSpec((F, MAP), lambda i: (0, 0)),
                  pl.BlockSpec((_RB, 16), lambda i: (cblk0 + i, 0)),
                  pl.BlockSpec((_RB, F), lambda i: (_edge_clamp(i), 0)),
                  pl.BlockSpec((_RB, 16),
                               lambda i: (tblk0 + _edge_clamp(i), 0))],
        out_specs=[pl.BlockSpec((2, _RB, MAP), lambda i: (0, i, 0)),
                   pl.BlockSpec((_RB, F), lambda i: (_edge_clamp(i), 0))],
        out_shape=[jax.ShapeDtypeStruct((2, n, MAP), jnp.float32),
                   jax.ShapeDtypeStruct((NE, F), jnp.float32)],
    )(sl, sr, z2, b2.reshape(1, F), wtop, wbot, cnt, t, cnt)


_CB = 2048
_CG = NE // _CB
_MG = NE // _RB


def _finale_kernel(u_ref, hb_ref, ew_ref, eb_ref, nm_ref, nb_ref, cnt_ref,
                   o_ref, nmh_s, emh_s, s0_ref, s1_ref, d_ref):
    s = pl.program_id(0)

    @pl.when(s < _MG)
    def _():
        c = cnt_ref[:, 0:1]
        dinv = jnp.where(c > 0, 1.0 / c, 0.0)
        g2 = _leaky(dinv * u_ref[...] + hb_ref[...])
        em = (jnp.dot(g2, ew_ref[...], preferred_element_type=jnp.float32)
              + eb_ref[...])
        emh = em * (lax.rsqrt(jnp.sum(em * em, axis=1, keepdims=True))
                    * _LOG2E)
        emh_s[pl.ds(s * _RB, _RB), :] = emh.astype(jnp.bfloat16)
        nm = nm_ref[...] + nb_ref[...]
        nmh = nm * lax.rsqrt(jnp.sum(nm * nm, axis=1, keepdims=True))
        nmh_s[pl.ds(s * _RB, _RB), :] = nmh.astype(jnp.bfloat16)

    @pl.when(s >= _MG)
    def _():
        i = (s - _MG) // _CG
        j = lax.rem(s - _MG, _CG)
        a = nmh_s[pl.ds(i * _CB, _CB), :]
        b = emh_s[pl.ds(j * _CB, _CB), :]
        m = lax.dot_general(a, b, (((1,), (1,)), ((), ())),
                            preferred_element_type=jnp.float32)
        e = jnp.exp2(-jnp.abs(m))
        ones = jnp.ones((_CB,), jnp.float32)
        rs = lax.dot_general(e, ones, (((1,), (0,)), ((), ())),
                             preferred_element_type=jnp.float32
                             ).reshape(_CB // 128, 128)
        cs = lax.dot_general(ones, e, (((0,), (0,)), ((), ())),
                             preferred_element_type=jnp.float32
                             ).reshape(_CB // 128, 128)
        rsl = pl.ds(i * (_CB // 128), _CB // 128)
        csl = pl.ds(j * (_CB // 128), _CB // 128)

        @pl.when(j == 0)
        def _():
            s1_ref[rsl, :] = jnp.zeros((_CB // 128, 128), jnp.float32)

        @pl.when(i == 0)
        def _():
            s0_ref[csl, :] = jnp.zeros((_CB // 128, 128), jnp.float32)

        s1_ref[rsl, :] += rs
        s0_ref[csl, :] += cs

        @pl.when(i == j)
        def _():
            r = lax.broadcasted_iota(jnp.int32, (_CB, _CB), 0)
            c = lax.broadcasted_iota(jnp.int32, (_CB, _CB), 1)
            diag = jnp.sum(jnp.where(r == c, m, 0.0), axis=1)
            d_ref[rsl, :] = diag.reshape(_CB // 128, 128)

    @pl.when(s == _MG + _CG * _CG - 1)
    def _():
        o_ref[...] = (jnp.abs(d_ref[...]) * (1.0 / _LOG2E) - jnp.log(2.0)
                      + jnp.log(s0_ref[...] + s1_ref[...]))


def _finale(u2, hgc_b2, edge_w, edge_b, nmp, node_b, cnt, cblk0):
    out = pl.pallas_call(
        _finale_kernel,
        grid=(_MG + _CG * _CG,),
        in_specs=[pl.BlockSpec((_RB, F), lambda s: (_edge_clamp(s), 0)),
                  pl.BlockSpec((1, F), lambda s: (0, 0)),
                  pl.BlockSpec((F, MAP), lambda s: (0, 0)),
                  pl.BlockSpec((1, MAP), lambda s: (0, 0)),
                  pl.BlockSpec((_RB, MAP), lambda s: (_edge_clamp(s), 0)),
                  pl.BlockSpec((1, MAP), lambda s: (0, 0)),
                  pl.BlockSpec((_RB, 16), lambda s: (cblk0 + _edge_clamp(s), 0))],
        out_specs=pl.BlockSpec((NE // 128, 128), lambda s: (0, 0)),
        out_shape=jax.ShapeDtypeStruct((NE // 128, 128), jnp.float32),
        scratch_shapes=[pltpu.VMEM((NE, MAP), jnp.bfloat16),
                        pltpu.VMEM((NE, MAP), jnp.bfloat16),
                        pltpu.VMEM((NE // 128, 128), jnp.float32),
                        pltpu.VMEM((NE // 128, 128), jnp.float32),
                        pltpu.VMEM((NE // 128, 128), jnp.float32)],
    )(u2, hgc_b2.reshape(1, F), edge_w, edge_b.reshape(1, MAP),
      nmp, node_b.reshape(1, MAP), cnt)
    return out.reshape(NE)


def kernel(nodes_feature, edges_feature, edge_index, hyperedge_index,
           gcn_w1, gcn_b1, gcn_w2, gcn_b2,
           hgc_w1, hgc_b1, hgc_w2, hgc_b2,
           node_w, node_b, edge_w, edge_b):
    ei0 = edge_index[0]
    ei1 = edge_index[1]
    hi0 = hyperedge_index[0]
    hi1 = hyperedge_index[1]

    x1e = _mm(edges_feature, hgc_w1)

    ones_tab = jnp.ones((8, 16), jnp.float32)
    npad = 81920 - (2 * NNZ + NE)
    dump = _CNT_ROWS + (jnp.arange(npad, dtype=jnp.int32)
                        % (_pad128(_CNT_ROWS) - _CNT_ROWS))
    cdst = jnp.concatenate([hi0, NE + hi1, 2 * NE + ei1, dump])
    cnt, t1 = _sc_dual(
        dict(table=ones_tab, src=jnp.zeros_like(cdst), dst=cdst,
             n_out=_CNT_ROWS, const_rows=True, chunk=128),
        dict(table=x1e, src=hi0, dst=hi1, n_out=NH))
    dblk = 0
    bblk = NE // _RB
    gblk = 2 * NE // _RB

    z1 = _mm_dinv(nodes_feature, gcn_w1, cnt, gblk)

    z1h = z1.reshape(2 * NN, MAP)
    s1l, s1r = _sc_dual(
        dict(table=z1h, src=2 * ei0, dst=ei1, n_out=NN),
        dict(table=z1h, src=2 * ei0 + 1, dst=ei1, n_out=NN))

    z2, he1 = _gcn_layer2_z(s1l, s1r, z1, gcn_b1, gcn_w2, cnt, gblk,
                            t1, bblk)

    z2h = z2.reshape(2 * NN, MAP)
    u1, s2l = _sc_dual(
        dict(table=he1, src=hi1, dst=hi0, n_out=NE, chunk=64),
        dict(table=z2h, src=2 * ei0, dst=ei1, n_out=NN, chunk=16))

    x2e = _hyper_layer_out(u1, hgc_b1, hgc_w2, cnt, dblk)

    t2, s2r = _sc_dual(
        dict(table=x2e, src=hi0, dst=hi1, n_out=NH, chunk=64),
        dict(table=z2h, src=2 * ei0 + 1, dst=ei1, n_out=NN, chunk=16))

    pq, he2 = _pq(s2l, s2r, z2, gcn_b2, node_w[:F], node_w[F:], cnt, gblk,
                  t2, bblk)

    iota_e = jnp.arange(NE, dtype=jnp.int32)
    nmsrc = jnp.concatenate([ei0, NN + ei1])
    nmdst = jnp.concatenate([iota_e, iota_e])
    u2, nmp = _sc_dual(
        dict(table=he2, src=hi1, dst=hi0, n_out=NE),
        dict(table=pq.reshape(2 * NN, MAP), src=nmsrc, dst=nmdst, n_out=NE))

    return _finale(u2, hgc_b2, edge_w, edge_b, nmp, node_b, cnt, dblk)
